# Initial kernel scaffold; baseline (speedup 1.0000x reference)
import jax
import jax.numpy as jnp
from jax.experimental import pallas as pl


def kernel(x_a, edge_index_a, edge_attr_a, batch_a, enzyme_a, x_b, edge_index_b, edge_attr_b, batch_b, enzyme_b, params):
    raise NotImplementedError("write your pallas kernel here")


# speedup vs baseline: 2.3593x; 2.3593x over previous
import jax
import jax.numpy as jnp
from jax import lax
from jax.experimental import pallas as pl
from jax.experimental.pallas import tpu as pltpu
from jax.experimental.pallas import tpu_sc as plsc

N = 10000
E = 160000
B = 256
HEADS = 4
LA_W = 128
NW = 32
EK = 64
NCHUNKS = E // EK
TPW = -(-NCHUNKS // NW)
LEK = 40
LEPW = E // NW
LNCHUNK = LEPW // LEK
NRC = 125
CPS = 8
NDC = 125

_MESH = plsc.VectorSubcoreMesh(core_axis_name="c", subcore_axis_name="s")
_SC_PARAMS = pltpu.CompilerParams(needs_layout_passes=False)


def _sc_edge_body(xl_hbm, xr_hbm, ee_hbm, src_hbm, dst_hbm, att_hbm,
                  num_hbm, den_hbm,
                  srcv, dstv, xl_v, xr_v, ee_v, row_v, att_v,
                  den_l, acc, s1, s2, s3):
    cid = lax.axis_index("c")
    sid = lax.axis_index("s")
    wid = sid * 2 + cid
    zeros16 = jnp.zeros((16,), jnp.float32)
    iota16 = lax.broadcasted_iota(jnp.int32, (16,), 0)

    def zd(i, c):
        den_l[pl.ds(i * 16, 16)] = zeros16
        return c
    lax.fori_loop(0, N // 16, zd, 0)

    def zz(r, c):
        for j in range(8):
            row_v[r, pl.ds(j * 16, 16)] = zeros16
        return c
    lax.fori_loop(0, EK, zz, 0)
    for t in range(10):
        cz = sid * 10 + t

        @pl.when(cz < 156)
        def _():
            pltpu.sync_copy(row_v, acc.at[pl.ds(cz * 64, 64)])

        @pl.when(cz == 156)
        def _():
            pltpu.sync_copy(row_v.at[pl.ds(0, 16)], acc.at[pl.ds(9984, 16)])
    pltpu.sync_copy(att_hbm, att_v)
    plsc.subcore_barrier()

    def tloop(t, c):
        cidx = t * NW + wid

        @pl.when(cidx < NCHUNKS)
        def _():
            eb = cidx * EK
            pltpu.sync_copy(src_hbm.at[pl.ds(eb, EK)], srcv)
            pltpu.sync_copy(dst_hbm.at[pl.ds(eb, EK)], dstv)
            d1 = pltpu.async_copy(xl_hbm.at[srcv], xl_v, s1)
            d2 = pltpu.async_copy(xr_hbm.at[dstv], xr_v, s2)
            d3 = pltpu.async_copy(ee_hbm.at[pl.ds(eb, EK)], ee_v, s3)
            d1.wait()
            d2.wait()
            d3.wait()

            for g in range(EK // 16):
                rows = g * 16 + iota16

                def lbody(cc, lg):
                    col = jnp.full((16,), cc, jnp.int32)
                    gxl = plsc.load_gather(xl_v, [rows, col])
                    gxr = plsc.load_gather(xr_v, [rows, col])
                    gee = plsc.load_gather(ee_v, [rows, col])
                    z = gxl + gxr + gee
                    a = att_v[pl.ds(cc, 16)][0]
                    return lg + jnp.maximum(z, 0.2 * z) * a
                lg = lax.fori_loop(0, 128, lbody, jnp.zeros((16,), jnp.float32))
                ex = jnp.exp(lg)

                def sbody(cc, c2):
                    col = jnp.full((16,), cc, jnp.int32)
                    gxl = plsc.load_gather(xl_v, [rows, col])
                    plsc.store_scatter(row_v, [rows, col], gxl * ex)
                    return c2
                lax.fori_loop(0, 128, sbody, 0)
                dstw = dstv[pl.ds(g * 16, 16)]
                plsc.addupdate_scatter(den_l, [dstw], ex)

            pltpu.sync_copy(row_v, acc.at[dstv], add=True)
        return c
    lax.fori_loop(0, TPW, tloop, 0)

    plsc.subcore_barrier()
    for t in range(CPS):
        cz = sid * CPS + t

        @pl.when(cz < NRC)
        def _():
            sl = pl.ds(cz * 80, 80)
            pltpu.sync_copy(acc.at[sl], num_hbm.at[cid, sl])
    pltpu.sync_copy(den_l, den_hbm.at[pl.ds(wid * N, N)])


_sc_edge = pl.kernel(
    _sc_edge_body,
    out_type=(jax.ShapeDtypeStruct((2, N, 128), jnp.float32),
              jax.ShapeDtypeStruct((NW * N,), jnp.float32)),
    mesh=_MESH,
    compiler_params=_SC_PARAMS,
    scratch_types=[
        pltpu.VMEM((EK,), jnp.int32),
        pltpu.VMEM((EK,), jnp.int32),
        pltpu.VMEM((EK, 128), jnp.float32),
        pltpu.VMEM((EK, 128), jnp.float32),
        pltpu.VMEM((EK, 128), jnp.float32),
        pltpu.VMEM((EK, 128), jnp.float32),
        pltpu.VMEM((144,), jnp.float32),
        pltpu.VMEM((N,), jnp.float32),
        pltpu.VMEM_SHARED((N, 128), jnp.float32),
        pltpu.SemaphoreType.DMA,
        pltpu.SemaphoreType.DMA,
        pltpu.SemaphoreType.DMA,
    ],
)


def _sc_la_body(ea_hbm, dst_hbm, out_hbm, dstv, ea_v, row_v, zbuf, acc, s1):
    cid = lax.axis_index("c")
    sid = lax.axis_index("s")
    wid = sid * 2 + cid
    zeros16 = jnp.zeros((16,), jnp.float32)
    iota = lax.broadcasted_iota(jnp.int32, (16,), 0)
    onehot0 = jnp.where(iota == 0, 1.0, 0.0).astype(jnp.float32)

    def zrow(e, c):
        row_v[e, pl.ds(0, 16)] = onehot0
        for j in range(2, LA_W // 16):
            row_v[e, pl.ds(j * 16, 16)] = zeros16
        return c
    lax.fori_loop(0, LEK, zrow, 0)

    def zz(r, c):
        for j in range(LA_W // 16):
            zbuf[r, pl.ds(j * 16, 16)] = zeros16
        return c
    lax.fori_loop(0, 80, zz, 0)
    for t in range(CPS):
        cz = sid * CPS + t

        @pl.when(cz < NRC)
        def _():
            pltpu.sync_copy(zbuf, acc.at[pl.ds(cz * 80, 80)])
    plsc.subcore_barrier()

    base = wid * LEPW

    def chunk(i, c):
        eb = base + i * LEK
        pltpu.sync_copy(dst_hbm.at[pl.ds(eb, LEK)], dstv)
        d1 = pltpu.async_copy(ea_hbm.at[pl.ds(eb, LEK)], ea_v, s1)
        d1.wait()

        def sbody(e, c2):
            row_v[e, pl.ds(16, 16)] = ea_v[e, pl.ds(0, 16)]
            return c2
        lax.fori_loop(0, LEK, sbody, 0)

        pltpu.sync_copy(row_v, acc.at[dstv], add=True)
        return c
    lax.fori_loop(0, LNCHUNK, chunk, 0)

    plsc.subcore_barrier()
    for t in range(CPS):
        cz = sid * CPS + t

        @pl.when(cz < NRC)
        def _():
            sl = pl.ds(cz * 80, 80)
            pltpu.sync_copy(acc.at[sl], out_hbm.at[cid, sl])


_sc_la = pl.kernel(
    _sc_la_body,
    out_type=jax.ShapeDtypeStruct((2, N, LA_W), jnp.float32),
    mesh=_MESH,
    compiler_params=_SC_PARAMS,
    scratch_types=[
        pltpu.VMEM((LEK,), jnp.int32),
        pltpu.VMEM((LEK, 16), jnp.float32),
        pltpu.VMEM((LEK, LA_W), jnp.float32),
        pltpu.VMEM((80, LA_W), jnp.float32),
        pltpu.VMEM_SHARED((N, LA_W), jnp.float32),
        pltpu.SemaphoreType.DMA,
    ],
)


def _mm(xs, ws, bias, act, bm):
    M = xs[0].shape[0]
    Ks = [x.shape[1] for x in xs]
    Nc = ws[0].shape[1]
    nx = len(xs)
    has_b = bias is not None

    def body(*refs):
        o = refs[-1]
        acc = None
        for i in range(nx):
            p = jnp.dot(refs[i][...], refs[nx + i][...],
                        preferred_element_type=jnp.float32)
            acc = p if acc is None else acc + p
        if has_b:
            acc = acc + refs[2 * nx][...]
        if act == "relu":
            acc = jnp.maximum(acc, 0.0)
        o[...] = acc

    in_specs = ([pl.BlockSpec((bm, k), lambda i: (i, 0)) for k in Ks]
                + [pl.BlockSpec((k, Nc), lambda i: (0, 0)) for k in Ks])
    if has_b:
        in_specs.append(pl.BlockSpec((1, Nc), lambda i: (0, 0)))
    args = list(xs) + list(ws) + ([bias] if has_b else [])
    return pl.pallas_call(
        body, grid=(M // bm,), in_specs=in_specs,
        out_specs=pl.BlockSpec((bm, Nc), lambda i: (i, 0)),
        out_shape=jax.ShapeDtypeStruct((M, Nc), jnp.float32))(*args)


def _init(xl, xr, q, we_h, att, bm=200):
    with_q = q is not None

    def body(*refs):
        if with_q:
            xlr, xrr, qr, wer, attr, o_num, o_den = refs
        else:
            xlr, xrr, attr, o_num, o_den = refs
        z = xlr[...] + xrr[...]
        if with_q:
            qs = qr[0] + qr[1]
            cnt = jnp.maximum(qs[:, 0:1], 1.0)
            mea = qs[:, 16:32] / cnt
            z = z + jnp.dot(mea, wer[...], preferred_element_type=jnp.float32)
        m = jnp.maximum(z, 0.2 * z) * attr[...]
        ex = jnp.exp(jnp.sum(m, axis=1, keepdims=True))
        o_num[...] = ex * xlr[...]
        o_den[...] = ex

    in_specs = [pl.BlockSpec((bm, 128), lambda i: (i, 0)),
                pl.BlockSpec((bm, 128), lambda i: (i, 0))]
    args = [xl, xr]
    if with_q:
        in_specs += [pl.BlockSpec((2, bm, LA_W), lambda i: (0, i, 0)),
                     pl.BlockSpec((16, 128), lambda i: (0, 0))]
        args += [q, we_h]
    in_specs.append(pl.BlockSpec((1, 128), lambda i: (0, 0)))
    args.append(att)
    return pl.pallas_call(
        body, grid=(N // bm,), in_specs=in_specs,
        out_specs=[pl.BlockSpec((bm, 128), lambda i: (i, 0)),
                   pl.BlockSpec((bm, 1), lambda i: (i, 0))],
        out_shape=[jax.ShapeDtypeStruct((N, 128), jnp.float32),
                   jax.ShapeDtypeStruct((N, 1), jnp.float32)])(*args)


def _fin(init_num, init_den, part_num, den4, bias, bm=200):
    def body(inr, idr, pr, ddr, br, o):
        tot = inr[...] + pr[0] + pr[1]
        d = jnp.sum(ddr[:, 0, 0, :], axis=0)
        ii = lax.broadcasted_iota(jnp.int32, (bm, bm), 0)
        jj = lax.broadcasted_iota(jnp.int32, (bm, bm), 1)
        dmat = jnp.where(ii == jj, d[None, :], 0.0)
        dcol = jnp.sum(dmat, axis=1, keepdims=True) + idr[...] + 1e-16
        out = tot / dcol + br[...]
        o[...] = jnp.where(out > 0, out, jnp.exp(out) - 1.0)

    nb = N // bm
    return pl.pallas_call(
        body, grid=(nb,),
        in_specs=[pl.BlockSpec((bm, 128), lambda i: (i, 0)),
                  pl.BlockSpec((bm, 1), lambda i: (i, 0)),
                  pl.BlockSpec((2, bm, 128), lambda i: (0, i, 0)),
                  pl.BlockSpec((NW, 1, 1, bm), lambda i: (0, i, 0, 0)),
                  pl.BlockSpec((1, 128), lambda i: (0, 0))],
        out_specs=pl.BlockSpec((bm, 128), lambda i: (i, 0)),
        out_shape=jax.ShapeDtypeStruct((N, 128), jnp.float32))(
            init_num, init_den, part_num, den4, bias)


def _pool(h, batch3, bm=200):
    nb = N // bm

    def body(hr, br, mean_o, mx_o, sum_s, cnt_s, mx_s):
        i = pl.program_id(0)
        bblk = br[0, 0, :]
        iota = lax.broadcasted_iota(jnp.int32, (B, bm), 0)
        maskf = (bblk[None, :] == iota).astype(jnp.float32)
        psum = jnp.dot(maskf, hr[...], preferred_element_type=jnp.float32)
        pcnt = jnp.dot(maskf, jnp.ones((bm, 128), jnp.float32),
                       preferred_element_type=jnp.float32)
        pmx = jnp.full((B, 128), -1e30, jnp.float32)
        for j in range(bm // 8):
            sub = hr[pl.ds(j * 8, 8), :]
            msk = maskf[:, j * 8:(j + 1) * 8]
            cand = jnp.where(msk[:, :, None] > 0, sub[None, :, :], -1e30)
            pmx = jnp.maximum(pmx, jnp.max(cand, axis=1))

        @pl.when(i == 0)
        def _():
            sum_s[...] = psum
            cnt_s[...] = pcnt
            mx_s[...] = pmx

        @pl.when(i > 0)
        def _():
            sum_s[...] += psum
            cnt_s[...] += pcnt
            mx_s[...] = jnp.maximum(mx_s[...], pmx)

        @pl.when(i == nb - 1)
        def _():
            c = cnt_s[...]
            mean_o[...] = sum_s[...] / jnp.maximum(c, 1.0)
            mx_o[...] = jnp.where(c > 0, mx_s[...], 0.0)

    return pl.pallas_call(
        body, grid=(nb,),
        in_specs=[pl.BlockSpec((bm, 128), lambda i: (i, 0)),
                  pl.BlockSpec((1, 1, bm), lambda i: (i, 0, 0))],
        out_specs=[pl.BlockSpec((B, 128), lambda i: (0, 0)),
                   pl.BlockSpec((B, 128), lambda i: (0, 0))],
        out_shape=[jax.ShapeDtypeStruct((B, 128), jnp.float32),
                   jax.ShapeDtypeStruct((B, 128), jnp.float32)],
        scratch_shapes=[pltpu.VMEM((B, 128), jnp.float32),
                        pltpu.VMEM((B, 128), jnp.float32),
                        pltpu.VMEM((B, 128), jnp.float32)])(h, batch3)


def _heads(mean_a, mx_a, mean_b, mx_b, enz_a, enz_b, w):
    def body(ma, xa, mb, xb, ea, eb,
             ghw1, ghb1, ghw2, ghb2, ghw3, ghb3,
             phw1, phb1, phw2, phb2,
             agw1a, agw1b, agb1, agw2, agb2,
             fin_o, al_o):
        va = jnp.concatenate([ma[...], xa[...]], axis=1)
        vb = jnp.concatenate([mb[...], xb[...]], axis=1)
        comb = jnp.concatenate([va + vb, jnp.abs(va - vb), va * vb], axis=1)
        ec = jnp.concatenate([ea[...] + eb[...], jnp.abs(ea[...] - eb[...]),
                              ea[...] * eb[...]], axis=1)
        dot = lambda a, b: jnp.dot(a, b, preferred_element_type=jnp.float32)
        h1 = jnp.maximum(dot(comb, ghw1[...]) + ghb1[...], 0.0)
        h2 = jnp.maximum(dot(h1, ghw2[...]) + ghb2[...], 0.0)
        gl = dot(h2, ghw3[...]) + ghb3[...]
        p1 = jnp.maximum(dot(ec, phw1[...]) + phb1[...], 0.0)
        plg = dot(p1, phw2[...]) + phb2[...]
        g = jnp.maximum(dot(comb, agw1a[...]) + dot(ec, agw1b[...])
                        + agb1[...], 0.0)
        al = 1.0 / (1.0 + jnp.exp(-(dot(g, agw2[...]) + agb2[...])))
        fin_o[...] = al * gl + (1.0 - al) * plg
        al_o[...] = al

    return pl.pallas_call(
        body,
        out_shape=[jax.ShapeDtypeStruct((B, 1), jnp.float32),
                   jax.ShapeDtypeStruct((B, 1), jnp.float32)],
    )(mean_a, mx_a, mean_b, mx_b, enz_a, enz_b, *w)


def kernel(x_a, edge_index_a, edge_attr_a, batch_a, enzyme_a,
           x_b, edge_index_b, edge_attr_b, batch_b, enzyme_b, params):
    p = params

    s_enc = p['enc_bn_g'] / jnp.sqrt(p['enc_bn_v'] + 1e-5)
    enc_w = p['enc_w'] * s_enc[None, :]
    enc_b = ((p['enc_b'] - p['enc_bn_m']) * s_enc + p['enc_bn_b']).reshape(1, -1)

    s_gh = p['gh_bn_g'] / jnp.sqrt(p['gh_bn_v'] + 1e-5)
    ghw1 = p['gh_w1'] * s_gh[None, :]
    ghb1 = ((p['gh_b1'] - p['gh_bn_m']) * s_gh + p['gh_bn_b']).reshape(1, -1)

    c1 = []
    for hh in range(HEADS):
        sl = slice(hh * 128, (hh + 1) * 128)
        c1.append(dict(
            wl=p['c1_wl'][:, sl], bl=p['c1_bl'][sl].reshape(1, -1),
            wr=p['c1_wr'][:, sl], br=p['c1_br'][sl].reshape(1, -1),
            we=p['c1_we'][:, sl], att=p['c1_att'][hh].reshape(1, -1),
            bias=p['c1_bias'][sl].reshape(1, -1)))
    c2_wl = [p['c2_wl'][i * 128:(i + 1) * 128, :] for i in range(HEADS)]
    c2_wr = [p['c2_wr'][i * 128:(i + 1) * 128, :] for i in range(HEADS)]

    def conv(xl, xr, eeh, q, we_h, att, bias, src, dst):
        init_num, init_den = _init(xl, xr, q, we_h, att)
        part_num, part_den = _sc_edge(xl, xr, eeh, src, dst,
                                      jnp.pad(att.reshape(-1), (0, 16)))
        den4 = part_den.reshape(NW, N // 200, 1, 200)
        return _fin(init_num, init_den, part_num, den4, bias)

    def arm(x, ei, ea, batch):
        src, dst = ei[0], ei[1]
        h0 = _mm([x], [enc_w], enc_b, "relu", 200)
        q = _sc_la(ea, dst)
        outs1 = []
        for hh in range(HEADS):
            c = c1[hh]
            xl = _mm([h0], [c['wl']], c['bl'], None, 200)
            xr = _mm([h0], [c['wr']], c['br'], None, 200)
            eeh = _mm([ea], [c['we']], None, None, 256)
            outs1.append(conv(xl, xr, eeh, q, c['we'], c['att'], c['bias'],
                              src, dst))
        xl2 = _mm(outs1, c2_wl, p['c2_bl'].reshape(1, -1), None, 200)
        xr2 = _mm(outs1, c2_wr, p['c2_br'].reshape(1, -1), None, 200)
        ee2 = _mm([ea], [p['c2_we']], None, None, 256)
        att2 = p['c2_att'].reshape(1, -1)
        h2 = conv(xl2, xr2, ee2, None, None, att2,
                  p['c2_bias'].reshape(1, -1), src, dst)
        batch3 = batch.reshape(N // 200, 1, 200)
        return _pool(h2, batch3)

    mean_a, mx_a = arm(x_a, edge_index_a, edge_attr_a, batch_a)
    mean_b, mx_b = arm(x_b, edge_index_b, edge_attr_b, batch_b)

    w = [ghw1, ghb1, p['gh_w2'], p['gh_b2'].reshape(1, -1),
         p['gh_w3'], p['gh_b3'].reshape(1, -1),
         p['ph_w1'], p['ph_b1'].reshape(1, -1),
         p['ph_w2'], p['ph_b2'].reshape(1, -1),
         p['ag_w1'][:768], p['ag_w1'][768:], p['ag_b1'].reshape(1, -1),
         p['ag_w2'], p['ag_b2'].reshape(1, -1)]
    final, alpha = _heads(mean_a, mx_a, mean_b, mx_b, enzyme_a, enzyme_b, w)
    return final, alpha


# speedup vs baseline: 2.4607x; 1.0430x over previous
import jax
import jax.numpy as jnp
from jax import lax
from jax.experimental import pallas as pl
from jax.experimental.pallas import tpu as pltpu
from jax.experimental.pallas import tpu_sc as plsc

N = 10000
E = 160000
B = 256
HEADS = 4
LA_W = 128
NW = 32
EK = 64
NCHUNKS = E // EK
TPW = -(-NCHUNKS // NW)
LEK = 40
LEPW = E // NW
LNCHUNK = LEPW // LEK
NRC = 125
CPS = 8
NDC = 125

_MESH = plsc.VectorSubcoreMesh(core_axis_name="c", subcore_axis_name="s")
_SC_PARAMS = pltpu.CompilerParams(needs_layout_passes=False)


def _sc_edge_body(xl_hbm, xr_hbm, ee_hbm, src_hbm, dst_hbm, att_hbm,
                  num_hbm, den_hbm,
                  srcv, dstv, xl_v, xr_v, ee_v, row_v, att_v,
                  den_l, acc, s1, s2, s3):
    cid = lax.axis_index("c")
    sid = lax.axis_index("s")
    wid = sid * 2 + cid
    zeros16 = jnp.zeros((16,), jnp.float32)
    iota16 = lax.broadcasted_iota(jnp.int32, (16,), 0)

    def zd(i, c):
        den_l[pl.ds(i * 16, 16)] = zeros16
        return c
    lax.fori_loop(0, N // 16, zd, 0)

    def zz(r, c):
        for j in range(8):
            row_v[r, pl.ds(j * 16, 16)] = zeros16
        return c
    lax.fori_loop(0, EK, zz, 0)
    for t in range(10):
        cz = sid * 10 + t

        @pl.when(cz < 156)
        def _():
            pltpu.sync_copy(row_v, acc.at[pl.ds(cz * 64, 64)])

        @pl.when(cz == 156)
        def _():
            pltpu.sync_copy(row_v.at[pl.ds(0, 16)], acc.at[pl.ds(9984, 16)])
    pltpu.sync_copy(att_hbm, att_v)
    plsc.subcore_barrier()

    def tloop(t, c):
        cidx = t * NW + wid

        @pl.when(cidx < NCHUNKS)
        def _():
            eb = cidx * EK
            pltpu.sync_copy(src_hbm.at[pl.ds(eb, EK)], srcv)
            pltpu.sync_copy(dst_hbm.at[pl.ds(eb, EK)], dstv)
            d1 = pltpu.async_copy(xl_hbm.at[srcv], xl_v, s1)
            d2 = pltpu.async_copy(xr_hbm.at[dstv], xr_v, s2)
            d3 = pltpu.async_copy(ee_hbm.at[pl.ds(eb, EK)], ee_v, s3)
            d1.wait()
            d2.wait()
            d3.wait()

            for g in range(EK // 16):
                rows = g * 16 + iota16

                def lbody(i, lg):
                    av = att_v[pl.ds(i * 4, 16)]
                    for u in range(4):
                        col = jnp.full((16,), i * 4 + u, jnp.int32)
                        gxl = plsc.load_gather(xl_v, [rows, col])
                        gxr = plsc.load_gather(xr_v, [rows, col])
                        gee = plsc.load_gather(ee_v, [rows, col])
                        z = gxl + gxr + gee
                        lg = lg + jnp.maximum(z, 0.2 * z) * av[u]
                    return lg
                lg = lax.fori_loop(0, 32, lbody, jnp.zeros((16,), jnp.float32))
                ex = jnp.exp(lg)

                def sbody(i, c2):
                    for u in range(4):
                        col = jnp.full((16,), i * 4 + u, jnp.int32)
                        gxl = plsc.load_gather(xl_v, [rows, col])
                        plsc.store_scatter(row_v, [rows, col], gxl * ex)
                    return c2
                lax.fori_loop(0, 32, sbody, 0)
                dstw = dstv[pl.ds(g * 16, 16)]
                plsc.addupdate_scatter(den_l, [dstw], ex)

            pltpu.sync_copy(row_v, acc.at[dstv], add=True)
        return c
    lax.fori_loop(0, TPW, tloop, 0)

    plsc.subcore_barrier()
    for t in range(CPS):
        cz = sid * CPS + t

        @pl.when(cz < NRC)
        def _():
            sl = pl.ds(cz * 80, 80)
            pltpu.sync_copy(acc.at[sl], num_hbm.at[cid, sl])
    pltpu.sync_copy(den_l, den_hbm.at[pl.ds(wid * N, N)])


_sc_edge = pl.kernel(
    _sc_edge_body,
    out_type=(jax.ShapeDtypeStruct((2, N, 128), jnp.float32),
              jax.ShapeDtypeStruct((NW * N,), jnp.float32)),
    mesh=_MESH,
    compiler_params=_SC_PARAMS,
    scratch_types=[
        pltpu.VMEM((EK,), jnp.int32),
        pltpu.VMEM((EK,), jnp.int32),
        pltpu.VMEM((EK, 128), jnp.float32),
        pltpu.VMEM((EK, 128), jnp.float32),
        pltpu.VMEM((EK, 128), jnp.float32),
        pltpu.VMEM((EK, 128), jnp.float32),
        pltpu.VMEM((144,), jnp.float32),
        pltpu.VMEM((N,), jnp.float32),
        pltpu.VMEM_SHARED((N, 128), jnp.float32),
        pltpu.SemaphoreType.DMA,
        pltpu.SemaphoreType.DMA,
        pltpu.SemaphoreType.DMA,
    ],
)


def _sc_la_body(ea_hbm, dst_hbm, out_hbm, dstv, ea_v, row_v, zbuf, acc, s1):
    cid = lax.axis_index("c")
    sid = lax.axis_index("s")
    wid = sid * 2 + cid
    zeros16 = jnp.zeros((16,), jnp.float32)
    iota = lax.broadcasted_iota(jnp.int32, (16,), 0)
    onehot0 = jnp.where(iota == 0, 1.0, 0.0).astype(jnp.float32)

    def zrow(e, c):
        row_v[e, pl.ds(0, 16)] = onehot0
        for j in range(2, LA_W // 16):
            row_v[e, pl.ds(j * 16, 16)] = zeros16
        return c
    lax.fori_loop(0, LEK, zrow, 0)

    def zz(r, c):
        for j in range(LA_W // 16):
            zbuf[r, pl.ds(j * 16, 16)] = zeros16
        return c
    lax.fori_loop(0, 80, zz, 0)
    for t in range(CPS):
        cz = sid * CPS + t

        @pl.when(cz < NRC)
        def _():
            pltpu.sync_copy(zbuf, acc.at[pl.ds(cz * 80, 80)])
    plsc.subcore_barrier()

    base = wid * LEPW

    def chunk(i, c):
        eb = base + i * LEK
        pltpu.sync_copy(dst_hbm.at[pl.ds(eb, LEK)], dstv)
        d1 = pltpu.async_copy(ea_hbm.at[pl.ds(eb, LEK)], ea_v, s1)
        d1.wait()

        def sbody(e, c2):
            row_v[e, pl.ds(16, 16)] = ea_v[e, pl.ds(0, 16)]
            return c2
        lax.fori_loop(0, LEK, sbody, 0)

        pltpu.sync_copy(row_v, acc.at[dstv], add=True)
        return c
    lax.fori_loop(0, LNCHUNK, chunk, 0)

    plsc.subcore_barrier()
    for t in range(CPS):
        cz = sid * CPS + t

        @pl.when(cz < NRC)
        def _():
            sl = pl.ds(cz * 80, 80)
            pltpu.sync_copy(acc.at[sl], out_hbm.at[cid, sl])


_sc_la = pl.kernel(
    _sc_la_body,
    out_type=jax.ShapeDtypeStruct((2, N, LA_W), jnp.float32),
    mesh=_MESH,
    compiler_params=_SC_PARAMS,
    scratch_types=[
        pltpu.VMEM((LEK,), jnp.int32),
        pltpu.VMEM((LEK, 16), jnp.float32),
        pltpu.VMEM((LEK, LA_W), jnp.float32),
        pltpu.VMEM((80, LA_W), jnp.float32),
        pltpu.VMEM_SHARED((N, LA_W), jnp.float32),
        pltpu.SemaphoreType.DMA,
    ],
)


def _mm(xs, ws, bias, act, bm):
    M = xs[0].shape[0]
    Ks = [x.shape[1] for x in xs]
    Nc = ws[0].shape[1]
    nx = len(xs)
    has_b = bias is not None

    def body(*refs):
        o = refs[-1]
        acc = None
        for i in range(nx):
            p = jnp.dot(refs[i][...], refs[nx + i][...],
                        preferred_element_type=jnp.float32)
            acc = p if acc is None else acc + p
        if has_b:
            acc = acc + refs[2 * nx][...]
        if act == "relu":
            acc = jnp.maximum(acc, 0.0)
        o[...] = acc

    in_specs = ([pl.BlockSpec((bm, k), lambda i: (i, 0)) for k in Ks]
                + [pl.BlockSpec((k, Nc), lambda i: (0, 0)) for k in Ks])
    if has_b:
        in_specs.append(pl.BlockSpec((1, Nc), lambda i: (0, 0)))
    args = list(xs) + list(ws) + ([bias] if has_b else [])
    return pl.pallas_call(
        body, grid=(M // bm,), in_specs=in_specs,
        out_specs=pl.BlockSpec((bm, Nc), lambda i: (i, 0)),
        out_shape=jax.ShapeDtypeStruct((M, Nc), jnp.float32))(*args)


def _init(xl, xr, q, we_h, att, bm=200):
    with_q = q is not None

    def body(*refs):
        if with_q:
            xlr, xrr, qr, wer, attr, o_num, o_den = refs
        else:
            xlr, xrr, attr, o_num, o_den = refs
        z = xlr[...] + xrr[...]
        if with_q:
            qs = qr[0] + qr[1]
            cnt = jnp.maximum(qs[:, 0:1], 1.0)
            mea = qs[:, 16:32] / cnt
            z = z + jnp.dot(mea, wer[...], preferred_element_type=jnp.float32)
        m = jnp.maximum(z, 0.2 * z) * attr[...]
        ex = jnp.exp(jnp.sum(m, axis=1, keepdims=True))
        o_num[...] = ex * xlr[...]
        o_den[...] = ex

    in_specs = [pl.BlockSpec((bm, 128), lambda i: (i, 0)),
                pl.BlockSpec((bm, 128), lambda i: (i, 0))]
    args = [xl, xr]
    if with_q:
        in_specs += [pl.BlockSpec((2, bm, LA_W), lambda i: (0, i, 0)),
                     pl.BlockSpec((16, 128), lambda i: (0, 0))]
        args += [q, we_h]
    in_specs.append(pl.BlockSpec((1, 128), lambda i: (0, 0)))
    args.append(att)
    return pl.pallas_call(
        body, grid=(N // bm,), in_specs=in_specs,
        out_specs=[pl.BlockSpec((bm, 128), lambda i: (i, 0)),
                   pl.BlockSpec((bm, 1), lambda i: (i, 0))],
        out_shape=[jax.ShapeDtypeStruct((N, 128), jnp.float32),
                   jax.ShapeDtypeStruct((N, 1), jnp.float32)])(*args)


def _fin(init_num, init_den, part_num, den4, bias, bm=200):
    def body(inr, idr, pr, ddr, br, o):
        tot = inr[...] + pr[0] + pr[1]
        d = jnp.sum(ddr[:, 0, 0, :], axis=0)
        ii = lax.broadcasted_iota(jnp.int32, (bm, bm), 0)
        jj = lax.broadcasted_iota(jnp.int32, (bm, bm), 1)
        dmat = jnp.where(ii == jj, d[None, :], 0.0)
        dcol = jnp.sum(dmat, axis=1, keepdims=True) + idr[...] + 1e-16
        out = tot / dcol + br[...]
        o[...] = jnp.where(out > 0, out, jnp.exp(out) - 1.0)

    nb = N // bm
    return pl.pallas_call(
        body, grid=(nb,),
        in_specs=[pl.BlockSpec((bm, 128), lambda i: (i, 0)),
                  pl.BlockSpec((bm, 1), lambda i: (i, 0)),
                  pl.BlockSpec((2, bm, 128), lambda i: (0, i, 0)),
                  pl.BlockSpec((NW, 1, 1, bm), lambda i: (0, i, 0, 0)),
                  pl.BlockSpec((1, 128), lambda i: (0, 0))],
        out_specs=pl.BlockSpec((bm, 128), lambda i: (i, 0)),
        out_shape=jax.ShapeDtypeStruct((N, 128), jnp.float32))(
            init_num, init_den, part_num, den4, bias)


def _pool(h, batch3, bm=200):
    nb = N // bm

    def body(hr, br, mean_o, mx_o, sum_s, cnt_s, mx_s):
        i = pl.program_id(0)
        bblk = br[0, 0, :]
        iota = lax.broadcasted_iota(jnp.int32, (B, bm), 0)
        maskf = (bblk[None, :] == iota).astype(jnp.float32)
        psum = jnp.dot(maskf, hr[...], preferred_element_type=jnp.float32)
        pcnt = jnp.dot(maskf, jnp.ones((bm, 128), jnp.float32),
                       preferred_element_type=jnp.float32)
        pmx = jnp.full((B, 128), -1e30, jnp.float32)
        for j in range(bm // 8):
            sub = hr[pl.ds(j * 8, 8), :]
            msk = maskf[:, j * 8:(j + 1) * 8]
            cand = jnp.where(msk[:, :, None] > 0, sub[None, :, :], -1e30)
            pmx = jnp.maximum(pmx, jnp.max(cand, axis=1))

        @pl.when(i == 0)
        def _():
            sum_s[...] = psum
            cnt_s[...] = pcnt
            mx_s[...] = pmx

        @pl.when(i > 0)
        def _():
            sum_s[...] += psum
            cnt_s[...] += pcnt
            mx_s[...] = jnp.maximum(mx_s[...], pmx)

        @pl.when(i == nb - 1)
        def _():
            c = cnt_s[...]
            mean_o[...] = sum_s[...] / jnp.maximum(c, 1.0)
            mx_o[...] = jnp.where(c > 0, mx_s[...], 0.0)

    return pl.pallas_call(
        body, grid=(nb,),
        in_specs=[pl.BlockSpec((bm, 128), lambda i: (i, 0)),
                  pl.BlockSpec((1, 1, bm), lambda i: (i, 0, 0))],
        out_specs=[pl.BlockSpec((B, 128), lambda i: (0, 0)),
                   pl.BlockSpec((B, 128), lambda i: (0, 0))],
        out_shape=[jax.ShapeDtypeStruct((B, 128), jnp.float32),
                   jax.ShapeDtypeStruct((B, 128), jnp.float32)],
        scratch_shapes=[pltpu.VMEM((B, 128), jnp.float32),
                        pltpu.VMEM((B, 128), jnp.float32),
                        pltpu.VMEM((B, 128), jnp.float32)])(h, batch3)


def _heads(mean_a, mx_a, mean_b, mx_b, enz_a, enz_b, w):
    def body(ma, xa, mb, xb, ea, eb,
             ghw1, ghb1, ghw2, ghb2, ghw3, ghb3,
             phw1, phb1, phw2, phb2,
             agw1a, agw1b, agb1, agw2, agb2,
             fin_o, al_o):
        va = jnp.concatenate([ma[...], xa[...]], axis=1)
        vb = jnp.concatenate([mb[...], xb[...]], axis=1)
        comb = jnp.concatenate([va + vb, jnp.abs(va - vb), va * vb], axis=1)
        ec = jnp.concatenate([ea[...] + eb[...], jnp.abs(ea[...] - eb[...]),
                              ea[...] * eb[...]], axis=1)
        dot = lambda a, b: jnp.dot(a, b, preferred_element_type=jnp.float32)
        h1 = jnp.maximum(dot(comb, ghw1[...]) + ghb1[...], 0.0)
        h2 = jnp.maximum(dot(h1, ghw2[...]) + ghb2[...], 0.0)
        gl = dot(h2, ghw3[...]) + ghb3[...]
        p1 = jnp.maximum(dot(ec, phw1[...]) + phb1[...], 0.0)
        plg = dot(p1, phw2[...]) + phb2[...]
        g = jnp.maximum(dot(comb, agw1a[...]) + dot(ec, agw1b[...])
                        + agb1[...], 0.0)
        al = 1.0 / (1.0 + jnp.exp(-(dot(g, agw2[...]) + agb2[...])))
        fin_o[...] = al * gl + (1.0 - al) * plg
        al_o[...] = al

    return pl.pallas_call(
        body,
        out_shape=[jax.ShapeDtypeStruct((B, 1), jnp.float32),
                   jax.ShapeDtypeStruct((B, 1), jnp.float32)],
    )(mean_a, mx_a, mean_b, mx_b, enz_a, enz_b, *w)


def kernel(x_a, edge_index_a, edge_attr_a, batch_a, enzyme_a,
           x_b, edge_index_b, edge_attr_b, batch_b, enzyme_b, params):
    p = params

    s_enc = p['enc_bn_g'] / jnp.sqrt(p['enc_bn_v'] + 1e-5)
    enc_w = p['enc_w'] * s_enc[None, :]
    enc_b = ((p['enc_b'] - p['enc_bn_m']) * s_enc + p['enc_bn_b']).reshape(1, -1)

    s_gh = p['gh_bn_g'] / jnp.sqrt(p['gh_bn_v'] + 1e-5)
    ghw1 = p['gh_w1'] * s_gh[None, :]
    ghb1 = ((p['gh_b1'] - p['gh_bn_m']) * s_gh + p['gh_bn_b']).reshape(1, -1)

    c1 = []
    for hh in range(HEADS):
        sl = slice(hh * 128, (hh + 1) * 128)
        c1.append(dict(
            wl=p['c1_wl'][:, sl], bl=p['c1_bl'][sl].reshape(1, -1),
            wr=p['c1_wr'][:, sl], br=p['c1_br'][sl].reshape(1, -1),
            we=p['c1_we'][:, sl], att=p['c1_att'][hh].reshape(1, -1),
            bias=p['c1_bias'][sl].reshape(1, -1)))
    c2_wl = [p['c2_wl'][i * 128:(i + 1) * 128, :] for i in range(HEADS)]
    c2_wr = [p['c2_wr'][i * 128:(i + 1) * 128, :] for i in range(HEADS)]

    def conv(xl, xr, eeh, q, we_h, att, bias, src, dst):
        init_num, init_den = _init(xl, xr, q, we_h, att)
        part_num, part_den = _sc_edge(xl, xr, eeh, src, dst,
                                      jnp.pad(att.reshape(-1), (0, 16)))
        den4 = part_den.reshape(NW, N // 200, 1, 200)
        return _fin(init_num, init_den, part_num, den4, bias)

    def arm(x, ei, ea, batch):
        src, dst = ei[0], ei[1]
        h0 = _mm([x], [enc_w], enc_b, "relu", 200)
        q = _sc_la(ea, dst)
        outs1 = []
        for hh in range(HEADS):
            c = c1[hh]
            xl = _mm([h0], [c['wl']], c['bl'], None, 200)
            xr = _mm([h0], [c['wr']], c['br'], None, 200)
            eeh = _mm([ea], [c['we']], None, None, 256)
            outs1.append(conv(xl, xr, eeh, q, c['we'], c['att'], c['bias'],
                              src, dst))
        xl2 = _mm(outs1, c2_wl, p['c2_bl'].reshape(1, -1), None, 200)
        xr2 = _mm(outs1, c2_wr, p['c2_br'].reshape(1, -1), None, 200)
        ee2 = _mm([ea], [p['c2_we']], None, None, 256)
        att2 = p['c2_att'].reshape(1, -1)
        h2 = conv(xl2, xr2, ee2, None, None, att2,
                  p['c2_bias'].reshape(1, -1), src, dst)
        batch3 = batch.reshape(N // 200, 1, 200)
        return _pool(h2, batch3)

    mean_a, mx_a = arm(x_a, edge_index_a, edge_attr_a, batch_a)
    mean_b, mx_b = arm(x_b, edge_index_b, edge_attr_b, batch_b)

    w = [ghw1, ghb1, p['gh_w2'], p['gh_b2'].reshape(1, -1),
         p['gh_w3'], p['gh_b3'].reshape(1, -1),
         p['ph_w1'], p['ph_b1'].reshape(1, -1),
         p['ph_w2'], p['ph_b2'].reshape(1, -1),
         p['ag_w1'][:768], p['ag_w1'][768:], p['ag_b1'].reshape(1, -1),
         p['ag_w2'], p['ag_b2'].reshape(1, -1)]
    final, alpha = _heads(mean_a, mx_a, mean_b, mx_b, enzyme_a, enzyme_b, w)
    return final, alpha


# speedup vs baseline: 2.5451x; 1.0343x over previous
import jax
import jax.numpy as jnp
from jax import lax
from jax.experimental import pallas as pl
from jax.experimental.pallas import tpu as pltpu
from jax.experimental.pallas import tpu_sc as plsc

N = 10000
E = 160000
B = 256
HEADS = 4
LA_W = 128
NW = 32
EK = 64
NCHUNKS = E // EK
TPW = -(-NCHUNKS // NW)
LEK = 40
LEPW = E // NW
LNCHUNK = LEPW // LEK
NRC = 125
CPS = 8
NDC = 125

_MESH = plsc.VectorSubcoreMesh(core_axis_name="c", subcore_axis_name="s")
_SC_PARAMS = pltpu.CompilerParams(needs_layout_passes=False)


def _sc_edge_body(xl_hbm, xr_hbm, ee_hbm, src_hbm, dst_hbm, att_hbm,
                  num_hbm, den_hbm,
                  srcv, dstv, xl_v, xr_v, ee_v, row_v, att_v,
                  den_l, acc, s1, s2, s3):
    cid = lax.axis_index("c")
    sid = lax.axis_index("s")
    wid = sid * 2 + cid
    zeros16 = jnp.zeros((16,), jnp.float32)
    iota16 = lax.broadcasted_iota(jnp.int32, (16,), 0)

    def zd(i, c):
        den_l[pl.ds(i * 16, 16)] = zeros16
        return c
    lax.fori_loop(0, N // 16, zd, 0)

    def zz(r, c):
        for j in range(8):
            row_v[r, pl.ds(j * 16, 16)] = zeros16
        return c
    lax.fori_loop(0, EK, zz, 0)
    for t in range(10):
        cz = sid * 10 + t

        @pl.when(cz < 156)
        def _():
            pltpu.sync_copy(row_v, acc.at[pl.ds(cz * 64, 64)])

        @pl.when(cz == 156)
        def _():
            pltpu.sync_copy(row_v.at[pl.ds(0, 16)], acc.at[pl.ds(9984, 16)])
    pltpu.sync_copy(att_hbm, att_v)
    plsc.subcore_barrier()

    def tloop(t, c):
        cidx = t * NW + wid

        @pl.when(cidx < NCHUNKS)
        def _():
            eb = cidx * EK
            pltpu.sync_copy(src_hbm.at[pl.ds(eb, EK)], srcv)
            pltpu.sync_copy(dst_hbm.at[pl.ds(eb, EK)], dstv)
            d1 = pltpu.async_copy(xl_hbm.at[srcv], xl_v, s1)
            d2 = pltpu.async_copy(xr_hbm.at[dstv], xr_v, s2)
            d3 = pltpu.async_copy(ee_hbm.at[pl.ds(eb, EK)], ee_v, s3)
            d1.wait()
            d2.wait()
            d3.wait()

            for g in range(EK // 16):
                rows = g * 16 + iota16

                def lbody(i, lg):
                    av = att_v[pl.ds(i * 8, 16)]
                    for u in range(8):
                        col = jnp.full((16,), i * 8 + u, jnp.int32)
                        gxl = plsc.load_gather(xl_v, [rows, col])
                        gxr = plsc.load_gather(xr_v, [rows, col])
                        gee = plsc.load_gather(ee_v, [rows, col])
                        z = gxl + gxr + gee
                        lg = lg + jnp.maximum(z, 0.2 * z) * av[u]
                    return lg
                lg = lax.fori_loop(0, 16, lbody, jnp.zeros((16,), jnp.float32))
                ex = jnp.exp(lg)

                def sbody(i, c2):
                    for u in range(8):
                        col = jnp.full((16,), i * 8 + u, jnp.int32)
                        gxl = plsc.load_gather(xl_v, [rows, col])
                        plsc.store_scatter(row_v, [rows, col], gxl * ex)
                    return c2
                lax.fori_loop(0, 16, sbody, 0)
                dstw = dstv[pl.ds(g * 16, 16)]
                plsc.addupdate_scatter(den_l, [dstw], ex)

            pltpu.sync_copy(row_v, acc.at[dstv], add=True)
        return c
    lax.fori_loop(0, TPW, tloop, 0)

    plsc.subcore_barrier()
    for t in range(CPS):
        cz = sid * CPS + t

        @pl.when(cz < NRC)
        def _():
            sl = pl.ds(cz * 80, 80)
            pltpu.sync_copy(acc.at[sl], num_hbm.at[cid, sl])
    pltpu.sync_copy(den_l, den_hbm.at[pl.ds(wid * N, N)])


_sc_edge = pl.kernel(
    _sc_edge_body,
    out_type=(jax.ShapeDtypeStruct((2, N, 128), jnp.float32),
              jax.ShapeDtypeStruct((NW * N,), jnp.float32)),
    mesh=_MESH,
    compiler_params=_SC_PARAMS,
    scratch_types=[
        pltpu.VMEM((EK,), jnp.int32),
        pltpu.VMEM((EK,), jnp.int32),
        pltpu.VMEM((EK, 128), jnp.float32),
        pltpu.VMEM((EK, 128), jnp.float32),
        pltpu.VMEM((EK, 128), jnp.float32),
        pltpu.VMEM((EK, 128), jnp.float32),
        pltpu.VMEM((144,), jnp.float32),
        pltpu.VMEM((N,), jnp.float32),
        pltpu.VMEM_SHARED((N, 128), jnp.float32),
        pltpu.SemaphoreType.DMA,
        pltpu.SemaphoreType.DMA,
        pltpu.SemaphoreType.DMA,
    ],
)


def _sc_la_body(ea_hbm, dst_hbm, out_hbm, dstv, ea_v, row_v, zbuf, acc, s1):
    cid = lax.axis_index("c")
    sid = lax.axis_index("s")
    wid = sid * 2 + cid
    zeros16 = jnp.zeros((16,), jnp.float32)
    iota = lax.broadcasted_iota(jnp.int32, (16,), 0)
    onehot0 = jnp.where(iota == 0, 1.0, 0.0).astype(jnp.float32)

    def zrow(e, c):
        row_v[e, pl.ds(0, 16)] = onehot0
        for j in range(2, LA_W // 16):
            row_v[e, pl.ds(j * 16, 16)] = zeros16
        return c
    lax.fori_loop(0, LEK, zrow, 0)

    def zz(r, c):
        for j in range(LA_W // 16):
            zbuf[r, pl.ds(j * 16, 16)] = zeros16
        return c
    lax.fori_loop(0, 80, zz, 0)
    for t in range(CPS):
        cz = sid * CPS + t

        @pl.when(cz < NRC)
        def _():
            pltpu.sync_copy(zbuf, acc.at[pl.ds(cz * 80, 80)])
    plsc.subcore_barrier()

    base = wid * LEPW

    def chunk(i, c):
        eb = base + i * LEK
        pltpu.sync_copy(dst_hbm.at[pl.ds(eb, LEK)], dstv)
        d1 = pltpu.async_copy(ea_hbm.at[pl.ds(eb, LEK)], ea_v, s1)
        d1.wait()

        def sbody(e, c2):
            row_v[e, pl.ds(16, 16)] = ea_v[e, pl.ds(0, 16)]
            return c2
        lax.fori_loop(0, LEK, sbody, 0)

        pltpu.sync_copy(row_v, acc.at[dstv], add=True)
        return c
    lax.fori_loop(0, LNCHUNK, chunk, 0)

    plsc.subcore_barrier()
    for t in range(CPS):
        cz = sid * CPS + t

        @pl.when(cz < NRC)
        def _():
            sl = pl.ds(cz * 80, 80)
            pltpu.sync_copy(acc.at[sl], out_hbm.at[cid, sl])


_sc_la = pl.kernel(
    _sc_la_body,
    out_type=jax.ShapeDtypeStruct((2, N, LA_W), jnp.float32),
    mesh=_MESH,
    compiler_params=_SC_PARAMS,
    scratch_types=[
        pltpu.VMEM((LEK,), jnp.int32),
        pltpu.VMEM((LEK, 16), jnp.float32),
        pltpu.VMEM((LEK, LA_W), jnp.float32),
        pltpu.VMEM((80, LA_W), jnp.float32),
        pltpu.VMEM_SHARED((N, LA_W), jnp.float32),
        pltpu.SemaphoreType.DMA,
    ],
)


def _mm(xs, ws, bias, act, bm):
    M = xs[0].shape[0]
    Ks = [x.shape[1] for x in xs]
    Nc = ws[0].shape[1]
    nx = len(xs)
    has_b = bias is not None

    def body(*refs):
        o = refs[-1]
        acc = None
        for i in range(nx):
            p = jnp.dot(refs[i][...], refs[nx + i][...],
                        preferred_element_type=jnp.float32)
            acc = p if acc is None else acc + p
        if has_b:
            acc = acc + refs[2 * nx][...]
        if act == "relu":
            acc = jnp.maximum(acc, 0.0)
        o[...] = acc

    in_specs = ([pl.BlockSpec((bm, k), lambda i: (i, 0)) for k in Ks]
                + [pl.BlockSpec((k, Nc), lambda i: (0, 0)) for k in Ks])
    if has_b:
        in_specs.append(pl.BlockSpec((1, Nc), lambda i: (0, 0)))
    args = list(xs) + list(ws) + ([bias] if has_b else [])
    return pl.pallas_call(
        body, grid=(M // bm,), in_specs=in_specs,
        out_specs=pl.BlockSpec((bm, Nc), lambda i: (i, 0)),
        out_shape=jax.ShapeDtypeStruct((M, Nc), jnp.float32))(*args)


def _init(xl, xr, q, we_h, att, bm=200):
    with_q = q is not None

    def body(*refs):
        if with_q:
            xlr, xrr, qr, wer, attr, o_num, o_den = refs
        else:
            xlr, xrr, attr, o_num, o_den = refs
        z = xlr[...] + xrr[...]
        if with_q:
            qs = qr[0] + qr[1]
            cnt = jnp.maximum(qs[:, 0:1], 1.0)
            mea = qs[:, 16:32] / cnt
            z = z + jnp.dot(mea, wer[...], preferred_element_type=jnp.float32)
        m = jnp.maximum(z, 0.2 * z) * attr[...]
        ex = jnp.exp(jnp.sum(m, axis=1, keepdims=True))
        o_num[...] = ex * xlr[...]
        o_den[...] = ex

    in_specs = [pl.BlockSpec((bm, 128), lambda i: (i, 0)),
                pl.BlockSpec((bm, 128), lambda i: (i, 0))]
    args = [xl, xr]
    if with_q:
        in_specs += [pl.BlockSpec((2, bm, LA_W), lambda i: (0, i, 0)),
                     pl.BlockSpec((16, 128), lambda i: (0, 0))]
        args += [q, we_h]
    in_specs.append(pl.BlockSpec((1, 128), lambda i: (0, 0)))
    args.append(att)
    return pl.pallas_call(
        body, grid=(N // bm,), in_specs=in_specs,
        out_specs=[pl.BlockSpec((bm, 128), lambda i: (i, 0)),
                   pl.BlockSpec((bm, 1), lambda i: (i, 0))],
        out_shape=[jax.ShapeDtypeStruct((N, 128), jnp.float32),
                   jax.ShapeDtypeStruct((N, 1), jnp.float32)])(*args)


def _fin(init_num, init_den, part_num, den4, bias, bm=200):
    def body(inr, idr, pr, ddr, br, o):
        tot = inr[...] + pr[0] + pr[1]
        d = jnp.sum(ddr[:, 0, 0, :], axis=0)
        ii = lax.broadcasted_iota(jnp.int32, (bm, bm), 0)
        jj = lax.broadcasted_iota(jnp.int32, (bm, bm), 1)
        dmat = jnp.where(ii == jj, d[None, :], 0.0)
        dcol = jnp.sum(dmat, axis=1, keepdims=True) + idr[...] + 1e-16
        out = tot / dcol + br[...]
        o[...] = jnp.where(out > 0, out, jnp.exp(out) - 1.0)

    nb = N // bm
    return pl.pallas_call(
        body, grid=(nb,),
        in_specs=[pl.BlockSpec((bm, 128), lambda i: (i, 0)),
                  pl.BlockSpec((bm, 1), lambda i: (i, 0)),
                  pl.BlockSpec((2, bm, 128), lambda i: (0, i, 0)),
                  pl.BlockSpec((NW, 1, 1, bm), lambda i: (0, i, 0, 0)),
                  pl.BlockSpec((1, 128), lambda i: (0, 0))],
        out_specs=pl.BlockSpec((bm, 128), lambda i: (i, 0)),
        out_shape=jax.ShapeDtypeStruct((N, 128), jnp.float32))(
            init_num, init_den, part_num, den4, bias)


def _pool(h, batch3, bm=200):
    nb = N // bm

    def body(hr, br, mean_o, mx_o, sum_s, cnt_s, mx_s):
        i = pl.program_id(0)
        bblk = br[0, 0, :]
        iota = lax.broadcasted_iota(jnp.int32, (B, bm), 0)
        maskf = (bblk[None, :] == iota).astype(jnp.float32)
        psum = jnp.dot(maskf, hr[...], preferred_element_type=jnp.float32)
        pcnt = jnp.dot(maskf, jnp.ones((bm, 128), jnp.float32),
                       preferred_element_type=jnp.float32)
        pmx = jnp.full((B, 128), -1e30, jnp.float32)
        for j in range(bm // 8):
            sub = hr[pl.ds(j * 8, 8), :]
            msk = maskf[:, j * 8:(j + 1) * 8]
            cand = jnp.where(msk[:, :, None] > 0, sub[None, :, :], -1e30)
            pmx = jnp.maximum(pmx, jnp.max(cand, axis=1))

        @pl.when(i == 0)
        def _():
            sum_s[...] = psum
            cnt_s[...] = pcnt
            mx_s[...] = pmx

        @pl.when(i > 0)
        def _():
            sum_s[...] += psum
            cnt_s[...] += pcnt
            mx_s[...] = jnp.maximum(mx_s[...], pmx)

        @pl.when(i == nb - 1)
        def _():
            c = cnt_s[...]
            mean_o[...] = sum_s[...] / jnp.maximum(c, 1.0)
            mx_o[...] = jnp.where(c > 0, mx_s[...], 0.0)

    return pl.pallas_call(
        body, grid=(nb,),
        in_specs=[pl.BlockSpec((bm, 128), lambda i: (i, 0)),
                  pl.BlockSpec((1, 1, bm), lambda i: (i, 0, 0))],
        out_specs=[pl.BlockSpec((B, 128), lambda i: (0, 0)),
                   pl.BlockSpec((B, 128), lambda i: (0, 0))],
        out_shape=[jax.ShapeDtypeStruct((B, 128), jnp.float32),
                   jax.ShapeDtypeStruct((B, 128), jnp.float32)],
        scratch_shapes=[pltpu.VMEM((B, 128), jnp.float32),
                        pltpu.VMEM((B, 128), jnp.float32),
                        pltpu.VMEM((B, 128), jnp.float32)])(h, batch3)


def _heads(mean_a, mx_a, mean_b, mx_b, enz_a, enz_b, w):
    def body(ma, xa, mb, xb, ea, eb,
             ghw1, ghb1, ghw2, ghb2, ghw3, ghb3,
             phw1, phb1, phw2, phb2,
             agw1a, agw1b, agb1, agw2, agb2,
             fin_o, al_o):
        va = jnp.concatenate([ma[...], xa[...]], axis=1)
        vb = jnp.concatenate([mb[...], xb[...]], axis=1)
        comb = jnp.concatenate([va + vb, jnp.abs(va - vb), va * vb], axis=1)
        ec = jnp.concatenate([ea[...] + eb[...], jnp.abs(ea[...] - eb[...]),
                              ea[...] * eb[...]], axis=1)
        dot = lambda a, b: jnp.dot(a, b, preferred_element_type=jnp.float32)
        h1 = jnp.maximum(dot(comb, ghw1[...]) + ghb1[...], 0.0)
        h2 = jnp.maximum(dot(h1, ghw2[...]) + ghb2[...], 0.0)
        gl = dot(h2, ghw3[...]) + ghb3[...]
        p1 = jnp.maximum(dot(ec, phw1[...]) + phb1[...], 0.0)
        plg = dot(p1, phw2[...]) + phb2[...]
        g = jnp.maximum(dot(comb, agw1a[...]) + dot(ec, agw1b[...])
                        + agb1[...], 0.0)
        al = 1.0 / (1.0 + jnp.exp(-(dot(g, agw2[...]) + agb2[...])))
        fin_o[...] = al * gl + (1.0 - al) * plg
        al_o[...] = al

    return pl.pallas_call(
        body,
        out_shape=[jax.ShapeDtypeStruct((B, 1), jnp.float32),
                   jax.ShapeDtypeStruct((B, 1), jnp.float32)],
    )(mean_a, mx_a, mean_b, mx_b, enz_a, enz_b, *w)


def kernel(x_a, edge_index_a, edge_attr_a, batch_a, enzyme_a,
           x_b, edge_index_b, edge_attr_b, batch_b, enzyme_b, params):
    p = params

    s_enc = p['enc_bn_g'] / jnp.sqrt(p['enc_bn_v'] + 1e-5)
    enc_w = p['enc_w'] * s_enc[None, :]
    enc_b = ((p['enc_b'] - p['enc_bn_m']) * s_enc + p['enc_bn_b']).reshape(1, -1)

    s_gh = p['gh_bn_g'] / jnp.sqrt(p['gh_bn_v'] + 1e-5)
    ghw1 = p['gh_w1'] * s_gh[None, :]
    ghb1 = ((p['gh_b1'] - p['gh_bn_m']) * s_gh + p['gh_bn_b']).reshape(1, -1)

    c1 = []
    for hh in range(HEADS):
        sl = slice(hh * 128, (hh + 1) * 128)
        c1.append(dict(
            wl=p['c1_wl'][:, sl], bl=p['c1_bl'][sl].reshape(1, -1),
            wr=p['c1_wr'][:, sl], br=p['c1_br'][sl].reshape(1, -1),
            we=p['c1_we'][:, sl], att=p['c1_att'][hh].reshape(1, -1),
            bias=p['c1_bias'][sl].reshape(1, -1)))
    c2_wl = [p['c2_wl'][i * 128:(i + 1) * 128, :] for i in range(HEADS)]
    c2_wr = [p['c2_wr'][i * 128:(i + 1) * 128, :] for i in range(HEADS)]

    def conv(xl, xr, eeh, q, we_h, att, bias, src, dst):
        init_num, init_den = _init(xl, xr, q, we_h, att)
        part_num, part_den = _sc_edge(xl, xr, eeh, src, dst,
                                      jnp.pad(att.reshape(-1), (0, 16)))
        den4 = part_den.reshape(NW, N // 200, 1, 200)
        return _fin(init_num, init_den, part_num, den4, bias)

    def arm(x, ei, ea, batch):
        src, dst = ei[0], ei[1]
        h0 = _mm([x], [enc_w], enc_b, "relu", 200)
        q = _sc_la(ea, dst)
        outs1 = []
        for hh in range(HEADS):
            c = c1[hh]
            xl = _mm([h0], [c['wl']], c['bl'], None, 200)
            xr = _mm([h0], [c['wr']], c['br'], None, 200)
            eeh = _mm([ea], [c['we']], None, None, 256)
            outs1.append(conv(xl, xr, eeh, q, c['we'], c['att'], c['bias'],
                              src, dst))
        xl2 = _mm(outs1, c2_wl, p['c2_bl'].reshape(1, -1), None, 200)
        xr2 = _mm(outs1, c2_wr, p['c2_br'].reshape(1, -1), None, 200)
        ee2 = _mm([ea], [p['c2_we']], None, None, 256)
        att2 = p['c2_att'].reshape(1, -1)
        h2 = conv(xl2, xr2, ee2, None, None, att2,
                  p['c2_bias'].reshape(1, -1), src, dst)
        batch3 = batch.reshape(N // 200, 1, 200)
        return _pool(h2, batch3)

    mean_a, mx_a = arm(x_a, edge_index_a, edge_attr_a, batch_a)
    mean_b, mx_b = arm(x_b, edge_index_b, edge_attr_b, batch_b)

    w = [ghw1, ghb1, p['gh_w2'], p['gh_b2'].reshape(1, -1),
         p['gh_w3'], p['gh_b3'].reshape(1, -1),
         p['ph_w1'], p['ph_b1'].reshape(1, -1),
         p['ph_w2'], p['ph_b2'].reshape(1, -1),
         p['ag_w1'][:768], p['ag_w1'][768:], p['ag_b1'].reshape(1, -1),
         p['ag_w2'], p['ag_b2'].reshape(1, -1)]
    final, alpha = _heads(mean_a, mx_a, mean_b, mx_b, enzyme_a, enzyme_b, w)
    return final, alpha


# speedup vs baseline: 7.9588x; 3.1271x over previous
import jax
import jax.numpy as jnp
from jax import lax
from jax.experimental import pallas as pl
from jax.experimental.pallas import tpu as pltpu
from jax.experimental.pallas import tpu_sc as plsc

N = 10000
E = 160000
B = 256
HEADS = 4
LA_W = 128
NW = 32
EK = 64
NCHUNKS = E // EK
TPW = -(-NCHUNKS // NW)
LEK = 40
LEPW = E // NW
LNCHUNK = LEPW // LEK
NRC = 125
CPS = 8
NDC = 125

_MESH = plsc.VectorSubcoreMesh(core_axis_name="c", subcore_axis_name="s")
_SC_PARAMS = pltpu.CompilerParams(needs_layout_passes=False)


def _sc_edge_body(xl_hbm, xr_hbm, ee_hbm, src_hbm, dst_hbm, att_hbm,
                  num_hbm, den_hbm,
                  srcv, dstv, xl_v, xr_v, ee_v, row_v, att_v,
                  pbuf, exbuf, den_l, acc, s1, s2, s3):
    cid = lax.axis_index("c")
    sid = lax.axis_index("s")
    wid = sid * 2 + cid
    zeros16 = jnp.zeros((16,), jnp.float32)
    iota16 = lax.broadcasted_iota(jnp.int32, (16,), 0)

    def zd(i, c):
        den_l[pl.ds(i * 16, 16)] = zeros16
        return c
    lax.fori_loop(0, N // 16, zd, 0)

    def zz(r, c):
        for j in range(8):
            row_v[r, pl.ds(j * 16, 16)] = zeros16
        return c
    lax.fori_loop(0, EK, zz, 0)
    for t in range(10):
        cz = sid * 10 + t

        @pl.when(cz < 156)
        def _():
            pltpu.sync_copy(row_v, acc.at[pl.ds(cz * 64, 64)])

        @pl.when(cz == 156)
        def _():
            pltpu.sync_copy(row_v.at[pl.ds(0, 16)], acc.at[pl.ds(9984, 16)])
    pltpu.sync_copy(att_hbm, att_v)
    plsc.subcore_barrier()
    att8 = [att_v[pl.ds(j * 16, 16)] for j in range(8)]

    def tloop(t, c):
        cidx = t * NW + wid

        @pl.when(cidx < NCHUNKS)
        def _():
            eb = cidx * EK
            pltpu.sync_copy(src_hbm.at[pl.ds(eb, EK)], srcv)
            pltpu.sync_copy(dst_hbm.at[pl.ds(eb, EK)], dstv)
            d1 = pltpu.async_copy(xl_hbm.at[srcv], xl_v, s1)
            d2 = pltpu.async_copy(xr_hbm.at[dstv], xr_v, s2)
            d3 = pltpu.async_copy(ee_hbm.at[pl.ds(eb, EK)], ee_v, s3)
            d1.wait()
            d2.wait()
            d3.wait()

            for g in range(EK // 16):
                def rbody(l, c2):
                    e = g * 16 + l
                    t8 = None
                    for j in range(8):
                        sl8 = pl.ds(j * 16, 16)
                        z = xl_v[e, sl8] + xr_v[e, sl8] + ee_v[e, sl8]
                        m = jnp.maximum(z, 0.2 * z) * att8[j]
                        t8 = m if t8 is None else t8 + m
                    pbuf[l, pl.ds(0, 16)] = t8
                    return c2
                lax.fori_loop(0, 16, rbody, 0)
                lg = jnp.zeros((16,), jnp.float32)
                for cc in range(16):
                    colv = plsc.load_gather(
                        pbuf, [iota16, jnp.full((16,), cc, jnp.int32)])
                    lg = lg + colv
                ex = jnp.exp(lg)
                exbuf[pl.ds(0, 16)] = ex

                def scbody(l, c2):
                    e = g * 16 + l
                    s = exbuf[pl.ds(l, 16)][0]
                    for j in range(8):
                        sl8 = pl.ds(j * 16, 16)
                        row_v[e, sl8] = xl_v[e, sl8] * s
                    return c2
                lax.fori_loop(0, 16, scbody, 0)
                dstw = dstv[pl.ds(g * 16, 16)]
                plsc.addupdate_scatter(den_l, [dstw], ex)

            pltpu.sync_copy(row_v, acc.at[dstv], add=True)
        return c
    lax.fori_loop(0, TPW, tloop, 0)

    plsc.subcore_barrier()
    for t in range(CPS):
        cz = sid * CPS + t

        @pl.when(cz < NRC)
        def _():
            sl = pl.ds(cz * 80, 80)
            pltpu.sync_copy(acc.at[sl], num_hbm.at[cid, sl])
    pltpu.sync_copy(den_l, den_hbm.at[pl.ds(wid * N, N)])


_sc_edge = pl.kernel(
    _sc_edge_body,
    out_type=(jax.ShapeDtypeStruct((2, N, 128), jnp.float32),
              jax.ShapeDtypeStruct((NW * N,), jnp.float32)),
    mesh=_MESH,
    compiler_params=_SC_PARAMS,
    scratch_types=[
        pltpu.VMEM((EK,), jnp.int32),
        pltpu.VMEM((EK,), jnp.int32),
        pltpu.VMEM((EK, 128), jnp.float32),
        pltpu.VMEM((EK, 128), jnp.float32),
        pltpu.VMEM((EK, 128), jnp.float32),
        pltpu.VMEM((EK, 128), jnp.float32),
        pltpu.VMEM((144,), jnp.float32),
        pltpu.VMEM((16, 16), jnp.float32),
        pltpu.VMEM((32,), jnp.float32),
        pltpu.VMEM((N,), jnp.float32),
        pltpu.VMEM_SHARED((N, 128), jnp.float32),
        pltpu.SemaphoreType.DMA,
        pltpu.SemaphoreType.DMA,
        pltpu.SemaphoreType.DMA,
    ],
)


def _sc_la_body(ea_hbm, dst_hbm, out_hbm, dstv, ea_v, row_v, zbuf, acc, s1):
    cid = lax.axis_index("c")
    sid = lax.axis_index("s")
    wid = sid * 2 + cid
    zeros16 = jnp.zeros((16,), jnp.float32)
    iota = lax.broadcasted_iota(jnp.int32, (16,), 0)
    onehot0 = jnp.where(iota == 0, 1.0, 0.0).astype(jnp.float32)

    def zrow(e, c):
        row_v[e, pl.ds(0, 16)] = onehot0
        for j in range(2, LA_W // 16):
            row_v[e, pl.ds(j * 16, 16)] = zeros16
        return c
    lax.fori_loop(0, LEK, zrow, 0)

    def zz(r, c):
        for j in range(LA_W // 16):
            zbuf[r, pl.ds(j * 16, 16)] = zeros16
        return c
    lax.fori_loop(0, 80, zz, 0)
    for t in range(CPS):
        cz = sid * CPS + t

        @pl.when(cz < NRC)
        def _():
            pltpu.sync_copy(zbuf, acc.at[pl.ds(cz * 80, 80)])
    plsc.subcore_barrier()

    base = wid * LEPW

    def chunk(i, c):
        eb = base + i * LEK
        pltpu.sync_copy(dst_hbm.at[pl.ds(eb, LEK)], dstv)
        d1 = pltpu.async_copy(ea_hbm.at[pl.ds(eb, LEK)], ea_v, s1)
        d1.wait()

        def sbody(e, c2):
            row_v[e, pl.ds(16, 16)] = ea_v[e, pl.ds(0, 16)]
            return c2
        lax.fori_loop(0, LEK, sbody, 0)

        pltpu.sync_copy(row_v, acc.at[dstv], add=True)
        return c
    lax.fori_loop(0, LNCHUNK, chunk, 0)

    plsc.subcore_barrier()
    for t in range(CPS):
        cz = sid * CPS + t

        @pl.when(cz < NRC)
        def _():
            sl = pl.ds(cz * 80, 80)
            pltpu.sync_copy(acc.at[sl], out_hbm.at[cid, sl])


_sc_la = pl.kernel(
    _sc_la_body,
    out_type=jax.ShapeDtypeStruct((2, N, LA_W), jnp.float32),
    mesh=_MESH,
    compiler_params=_SC_PARAMS,
    scratch_types=[
        pltpu.VMEM((LEK,), jnp.int32),
        pltpu.VMEM((LEK, 16), jnp.float32),
        pltpu.VMEM((LEK, LA_W), jnp.float32),
        pltpu.VMEM((80, LA_W), jnp.float32),
        pltpu.VMEM_SHARED((N, LA_W), jnp.float32),
        pltpu.SemaphoreType.DMA,
    ],
)


def _mm(xs, ws, bias, act, bm):
    M = xs[0].shape[0]
    Ks = [x.shape[1] for x in xs]
    Nc = ws[0].shape[1]
    nx = len(xs)
    has_b = bias is not None

    def body(*refs):
        o = refs[-1]
        acc = None
        for i in range(nx):
            p = jnp.dot(refs[i][...], refs[nx + i][...],
                        preferred_element_type=jnp.float32)
            acc = p if acc is None else acc + p
        if has_b:
            acc = acc + refs[2 * nx][...]
        if act == "relu":
            acc = jnp.maximum(acc, 0.0)
        o[...] = acc

    in_specs = ([pl.BlockSpec((bm, k), lambda i: (i, 0)) for k in Ks]
                + [pl.BlockSpec((k, Nc), lambda i: (0, 0)) for k in Ks])
    if has_b:
        in_specs.append(pl.BlockSpec((1, Nc), lambda i: (0, 0)))
    args = list(xs) + list(ws) + ([bias] if has_b else [])
    return pl.pallas_call(
        body, grid=(M // bm,), in_specs=in_specs,
        out_specs=pl.BlockSpec((bm, Nc), lambda i: (i, 0)),
        out_shape=jax.ShapeDtypeStruct((M, Nc), jnp.float32))(*args)


def _init(xl, xr, q, we_h, att, bm=200):
    with_q = q is not None

    def body(*refs):
        if with_q:
            xlr, xrr, qr, wer, attr, o_num, o_den = refs
        else:
            xlr, xrr, attr, o_num, o_den = refs
        z = xlr[...] + xrr[...]
        if with_q:
            qs = qr[0] + qr[1]
            cnt = jnp.maximum(qs[:, 0:1], 1.0)
            mea = qs[:, 16:32] / cnt
            z = z + jnp.dot(mea, wer[...], preferred_element_type=jnp.float32)
        m = jnp.maximum(z, 0.2 * z) * attr[...]
        ex = jnp.exp(jnp.sum(m, axis=1, keepdims=True))
        o_num[...] = ex * xlr[...]
        o_den[...] = ex

    in_specs = [pl.BlockSpec((bm, 128), lambda i: (i, 0)),
                pl.BlockSpec((bm, 128), lambda i: (i, 0))]
    args = [xl, xr]
    if with_q:
        in_specs += [pl.BlockSpec((2, bm, LA_W), lambda i: (0, i, 0)),
                     pl.BlockSpec((16, 128), lambda i: (0, 0))]
        args += [q, we_h]
    in_specs.append(pl.BlockSpec((1, 128), lambda i: (0, 0)))
    args.append(att)
    return pl.pallas_call(
        body, grid=(N // bm,), in_specs=in_specs,
        out_specs=[pl.BlockSpec((bm, 128), lambda i: (i, 0)),
                   pl.BlockSpec((bm, 1), lambda i: (i, 0))],
        out_shape=[jax.ShapeDtypeStruct((N, 128), jnp.float32),
                   jax.ShapeDtypeStruct((N, 1), jnp.float32)])(*args)


def _fin(init_num, init_den, part_num, den4, bias, bm=200):
    def body(inr, idr, pr, ddr, br, o):
        tot = inr[...] + pr[0] + pr[1]
        d = jnp.sum(ddr[:, 0, 0, :], axis=0)
        ii = lax.broadcasted_iota(jnp.int32, (bm, bm), 0)
        jj = lax.broadcasted_iota(jnp.int32, (bm, bm), 1)
        dmat = jnp.where(ii == jj, d[None, :], 0.0)
        dcol = jnp.sum(dmat, axis=1, keepdims=True) + idr[...] + 1e-16
        out = tot / dcol + br[...]
        o[...] = jnp.where(out > 0, out, jnp.exp(out) - 1.0)

    nb = N // bm
    return pl.pallas_call(
        body, grid=(nb,),
        in_specs=[pl.BlockSpec((bm, 128), lambda i: (i, 0)),
                  pl.BlockSpec((bm, 1), lambda i: (i, 0)),
                  pl.BlockSpec((2, bm, 128), lambda i: (0, i, 0)),
                  pl.BlockSpec((NW, 1, 1, bm), lambda i: (0, i, 0, 0)),
                  pl.BlockSpec((1, 128), lambda i: (0, 0))],
        out_specs=pl.BlockSpec((bm, 128), lambda i: (i, 0)),
        out_shape=jax.ShapeDtypeStruct((N, 128), jnp.float32))(
            init_num, init_den, part_num, den4, bias)


def _pool(h, batch3, bm=200):
    nb = N // bm

    def body(hr, br, mean_o, mx_o, sum_s, cnt_s, mx_s):
        i = pl.program_id(0)
        bblk = br[0, 0, :]
        iota = lax.broadcasted_iota(jnp.int32, (B, bm), 0)
        maskf = (bblk[None, :] == iota).astype(jnp.float32)
        psum = jnp.dot(maskf, hr[...], preferred_element_type=jnp.float32)
        pcnt = jnp.dot(maskf, jnp.ones((bm, 128), jnp.float32),
                       preferred_element_type=jnp.float32)
        pmx = jnp.full((B, 128), -1e30, jnp.float32)
        for j in range(bm // 8):
            sub = hr[pl.ds(j * 8, 8), :]
            msk = maskf[:, j * 8:(j + 1) * 8]
            cand = jnp.where(msk[:, :, None] > 0, sub[None, :, :], -1e30)
            pmx = jnp.maximum(pmx, jnp.max(cand, axis=1))

        @pl.when(i == 0)
        def _():
            sum_s[...] = psum
            cnt_s[...] = pcnt
            mx_s[...] = pmx

        @pl.when(i > 0)
        def _():
            sum_s[...] += psum
            cnt_s[...] += pcnt
            mx_s[...] = jnp.maximum(mx_s[...], pmx)

        @pl.when(i == nb - 1)
        def _():
            c = cnt_s[...]
            mean_o[...] = sum_s[...] / jnp.maximum(c, 1.0)
            mx_o[...] = jnp.where(c > 0, mx_s[...], 0.0)

    return pl.pallas_call(
        body, grid=(nb,),
        in_specs=[pl.BlockSpec((bm, 128), lambda i: (i, 0)),
                  pl.BlockSpec((1, 1, bm), lambda i: (i, 0, 0))],
        out_specs=[pl.BlockSpec((B, 128), lambda i: (0, 0)),
                   pl.BlockSpec((B, 128), lambda i: (0, 0))],
        out_shape=[jax.ShapeDtypeStruct((B, 128), jnp.float32),
                   jax.ShapeDtypeStruct((B, 128), jnp.float32)],
        scratch_shapes=[pltpu.VMEM((B, 128), jnp.float32),
                        pltpu.VMEM((B, 128), jnp.float32),
                        pltpu.VMEM((B, 128), jnp.float32)])(h, batch3)


def _heads(mean_a, mx_a, mean_b, mx_b, enz_a, enz_b, w):
    def body(ma, xa, mb, xb, ea, eb,
             ghw1, ghb1, ghw2, ghb2, ghw3, ghb3,
             phw1, phb1, phw2, phb2,
             agw1a, agw1b, agb1, agw2, agb2,
             fin_o, al_o):
        va = jnp.concatenate([ma[...], xa[...]], axis=1)
        vb = jnp.concatenate([mb[...], xb[...]], axis=1)
        comb = jnp.concatenate([va + vb, jnp.abs(va - vb), va * vb], axis=1)
        ec = jnp.concatenate([ea[...] + eb[...], jnp.abs(ea[...] - eb[...]),
                              ea[...] * eb[...]], axis=1)
        dot = lambda a, b: jnp.dot(a, b, preferred_element_type=jnp.float32)
        h1 = jnp.maximum(dot(comb, ghw1[...]) + ghb1[...], 0.0)
        h2 = jnp.maximum(dot(h1, ghw2[...]) + ghb2[...], 0.0)
        gl = dot(h2, ghw3[...]) + ghb3[...]
        p1 = jnp.maximum(dot(ec, phw1[...]) + phb1[...], 0.0)
        plg = dot(p1, phw2[...]) + phb2[...]
        g = jnp.maximum(dot(comb, agw1a[...]) + dot(ec, agw1b[...])
                        + agb1[...], 0.0)
        al = 1.0 / (1.0 + jnp.exp(-(dot(g, agw2[...]) + agb2[...])))
        fin_o[...] = al * gl + (1.0 - al) * plg
        al_o[...] = al

    return pl.pallas_call(
        body,
        out_shape=[jax.ShapeDtypeStruct((B, 1), jnp.float32),
                   jax.ShapeDtypeStruct((B, 1), jnp.float32)],
    )(mean_a, mx_a, mean_b, mx_b, enz_a, enz_b, *w)


def kernel(x_a, edge_index_a, edge_attr_a, batch_a, enzyme_a,
           x_b, edge_index_b, edge_attr_b, batch_b, enzyme_b, params):
    p = params

    s_enc = p['enc_bn_g'] / jnp.sqrt(p['enc_bn_v'] + 1e-5)
    enc_w = p['enc_w'] * s_enc[None, :]
    enc_b = ((p['enc_b'] - p['enc_bn_m']) * s_enc + p['enc_bn_b']).reshape(1, -1)

    s_gh = p['gh_bn_g'] / jnp.sqrt(p['gh_bn_v'] + 1e-5)
    ghw1 = p['gh_w1'] * s_gh[None, :]
    ghb1 = ((p['gh_b1'] - p['gh_bn_m']) * s_gh + p['gh_bn_b']).reshape(1, -1)

    c1 = []
    for hh in range(HEADS):
        sl = slice(hh * 128, (hh + 1) * 128)
        c1.append(dict(
            wl=p['c1_wl'][:, sl], bl=p['c1_bl'][sl].reshape(1, -1),
            wr=p['c1_wr'][:, sl], br=p['c1_br'][sl].reshape(1, -1),
            we=p['c1_we'][:, sl], att=p['c1_att'][hh].reshape(1, -1),
            bias=p['c1_bias'][sl].reshape(1, -1)))
    c2_wl = [p['c2_wl'][i * 128:(i + 1) * 128, :] for i in range(HEADS)]
    c2_wr = [p['c2_wr'][i * 128:(i + 1) * 128, :] for i in range(HEADS)]

    def conv(xl, xr, eeh, q, we_h, att, bias, src, dst):
        init_num, init_den = _init(xl, xr, q, we_h, att)
        part_num, part_den = _sc_edge(xl, xr, eeh, src, dst,
                                      jnp.pad(att.reshape(-1), (0, 16)))
        den4 = part_den.reshape(NW, N // 200, 1, 200)
        return _fin(init_num, init_den, part_num, den4, bias)

    def arm(x, ei, ea, batch):
        src, dst = ei[0], ei[1]
        h0 = _mm([x], [enc_w], enc_b, "relu", 200)
        q = _sc_la(ea, dst)
        outs1 = []
        for hh in range(HEADS):
            c = c1[hh]
            xl = _mm([h0], [c['wl']], c['bl'], None, 200)
            xr = _mm([h0], [c['wr']], c['br'], None, 200)
            eeh = _mm([ea], [c['we']], None, None, 256)
            outs1.append(conv(xl, xr, eeh, q, c['we'], c['att'], c['bias'],
                              src, dst))
        xl2 = _mm(outs1, c2_wl, p['c2_bl'].reshape(1, -1), None, 200)
        xr2 = _mm(outs1, c2_wr, p['c2_br'].reshape(1, -1), None, 200)
        ee2 = _mm([ea], [p['c2_we']], None, None, 256)
        att2 = p['c2_att'].reshape(1, -1)
        h2 = conv(xl2, xr2, ee2, None, None, att2,
                  p['c2_bias'].reshape(1, -1), src, dst)
        batch3 = batch.reshape(N // 200, 1, 200)
        return _pool(h2, batch3)

    mean_a, mx_a = arm(x_a, edge_index_a, edge_attr_a, batch_a)
    mean_b, mx_b = arm(x_b, edge_index_b, edge_attr_b, batch_b)

    w = [ghw1, ghb1, p['gh_w2'], p['gh_b2'].reshape(1, -1),
         p['gh_w3'], p['gh_b3'].reshape(1, -1),
         p['ph_w1'], p['ph_b1'].reshape(1, -1),
         p['ph_w2'], p['ph_b2'].reshape(1, -1),
         p['ag_w1'][:768], p['ag_w1'][768:], p['ag_b1'].reshape(1, -1),
         p['ag_w2'], p['ag_b2'].reshape(1, -1)]
    final, alpha = _heads(mean_a, mx_a, mean_b, mx_b, enzyme_a, enzyme_b, w)
    return final, alpha


# speedup vs baseline: 8.2949x; 1.0422x over previous
import jax
import jax.numpy as jnp
from jax import lax
from jax.experimental import pallas as pl
from jax.experimental.pallas import tpu as pltpu
from jax.experimental.pallas import tpu_sc as plsc

N = 10000
E = 160000
B = 256
HEADS = 4
LA_W = 128
NW = 32
EK = 64
NCHUNKS = E // EK
TPW = -(-NCHUNKS // NW)
LEK = 40
LEPW = E // NW
LNCHUNK = LEPW // LEK
NRC = 125
CPS = 8
NDC = 125

_MESH = plsc.VectorSubcoreMesh(core_axis_name="c", subcore_axis_name="s")
_SC_PARAMS = pltpu.CompilerParams(needs_layout_passes=False)


def _sc_edge_body(xl_hbm, xr_hbm, ee_hbm, src_hbm, dst_hbm, att_hbm,
                  num_hbm, den_hbm,
                  srcv, dstv, xl_v, xr_v, ee_v, row_v, att_v,
                  pbuf, exbuf, den_l, acc, s1, s2, s3):
    cid = lax.axis_index("c")
    sid = lax.axis_index("s")
    wid = sid * 2 + cid
    zeros16 = jnp.zeros((16,), jnp.float32)
    iota16 = lax.broadcasted_iota(jnp.int32, (16,), 0)

    def zd(i, c):
        den_l[pl.ds(i * 16, 16)] = zeros16
        return c
    lax.fori_loop(0, N // 16, zd, 0)

    def zz(r, c):
        for j in range(8):
            row_v[r, pl.ds(j * 16, 16)] = zeros16
        return c
    lax.fori_loop(0, EK, zz, 0)
    for t in range(10):
        cz = sid * 10 + t

        @pl.when(cz < 156)
        def _():
            pltpu.sync_copy(row_v, acc.at[pl.ds(cz * 64, 64)])

        @pl.when(cz == 156)
        def _():
            pltpu.sync_copy(row_v.at[pl.ds(0, 16)], acc.at[pl.ds(9984, 16)])
    pltpu.sync_copy(att_hbm, att_v)
    plsc.subcore_barrier()
    att8 = [att_v[pl.ds(j * 16, 16)] for j in range(8)]

    def tloop(t, c):
        cidx = t * NW + wid

        @pl.when(cidx < NCHUNKS)
        def _():
            eb = cidx * EK
            pltpu.sync_copy(src_hbm.at[pl.ds(eb, EK)], srcv)
            pltpu.sync_copy(dst_hbm.at[pl.ds(eb, EK)], dstv)
            d1 = pltpu.async_copy(xl_hbm.at[srcv], xl_v, s1)
            d2 = pltpu.async_copy(xr_hbm.at[dstv], xr_v, s2)
            d3 = pltpu.async_copy(ee_hbm.at[pl.ds(eb, EK)], ee_v, s3)
            d1.wait()
            d2.wait()
            d3.wait()

            for g in range(EK // 16):
                def rbody(l, c2):
                    e = g * 16 + l
                    t8 = None
                    for j in range(8):
                        sl8 = pl.ds(j * 16, 16)
                        z = xl_v[e, sl8] + xr_v[e, sl8] + ee_v[e, sl8]
                        m = jnp.maximum(z, 0.2 * z) * att8[j]
                        t8 = m if t8 is None else t8 + m
                    pbuf[l, pl.ds(0, 16)] = t8
                    return c2
                lax.fori_loop(0, 16, rbody, 0)
                lg = jnp.zeros((16,), jnp.float32)
                for cc in range(16):
                    colv = plsc.load_gather(
                        pbuf, [iota16, jnp.full((16,), cc, jnp.int32)])
                    lg = lg + colv
                ex = jnp.exp(lg)
                exbuf[pl.ds(0, 16)] = ex

                def scbody(l, c2):
                    e = g * 16 + l
                    s = exbuf[pl.ds(l, 16)][0]
                    for j in range(8):
                        sl8 = pl.ds(j * 16, 16)
                        row_v[e, sl8] = xl_v[e, sl8] * s
                    return c2
                lax.fori_loop(0, 16, scbody, 0)
                dstw = dstv[pl.ds(g * 16, 16)]
                plsc.addupdate_scatter(den_l, [dstw], ex)

            pltpu.sync_copy(row_v, acc.at[dstv], add=True)
        return c
    lax.fori_loop(0, TPW, tloop, 0)

    plsc.subcore_barrier()
    for t in range(CPS):
        cz = sid * CPS + t

        @pl.when(cz < NRC)
        def _():
            sl = pl.ds(cz * 80, 80)
            pltpu.sync_copy(acc.at[sl], num_hbm.at[cid, sl])
    pltpu.sync_copy(den_l, den_hbm.at[pl.ds(wid * N, N)])


_sc_edge = pl.kernel(
    _sc_edge_body,
    out_type=(jax.ShapeDtypeStruct((2, N, 128), jnp.float32),
              jax.ShapeDtypeStruct((NW * N,), jnp.float32)),
    mesh=_MESH,
    compiler_params=_SC_PARAMS,
    scratch_types=[
        pltpu.VMEM((EK,), jnp.int32),
        pltpu.VMEM((EK,), jnp.int32),
        pltpu.VMEM((EK, 128), jnp.float32),
        pltpu.VMEM((EK, 128), jnp.float32),
        pltpu.VMEM((EK, 128), jnp.float32),
        pltpu.VMEM((EK, 128), jnp.float32),
        pltpu.VMEM((144,), jnp.float32),
        pltpu.VMEM((16, 16), jnp.float32),
        pltpu.VMEM((32,), jnp.float32),
        pltpu.VMEM((N,), jnp.float32),
        pltpu.VMEM_SHARED((N, 128), jnp.float32),
        pltpu.SemaphoreType.DMA,
        pltpu.SemaphoreType.DMA,
        pltpu.SemaphoreType.DMA,
    ],
)


def _sc_la_body(ea_hbm, dst_hbm, out_hbm, dstv, ea_v, row_v, zbuf, acc, s1):
    cid = lax.axis_index("c")
    sid = lax.axis_index("s")
    wid = sid * 2 + cid
    zeros16 = jnp.zeros((16,), jnp.float32)
    iota = lax.broadcasted_iota(jnp.int32, (16,), 0)
    onehot0 = jnp.where(iota == 0, 1.0, 0.0).astype(jnp.float32)

    def zrow(e, c):
        row_v[e, pl.ds(0, 16)] = onehot0
        for j in range(2, LA_W // 16):
            row_v[e, pl.ds(j * 16, 16)] = zeros16
        return c
    lax.fori_loop(0, LEK, zrow, 0)

    def zz(r, c):
        for j in range(LA_W // 16):
            zbuf[r, pl.ds(j * 16, 16)] = zeros16
        return c
    lax.fori_loop(0, 80, zz, 0)
    for t in range(CPS):
        cz = sid * CPS + t

        @pl.when(cz < NRC)
        def _():
            pltpu.sync_copy(zbuf, acc.at[pl.ds(cz * 80, 80)])
    plsc.subcore_barrier()

    base = wid * LEPW

    def chunk(i, c):
        eb = base + i * LEK
        pltpu.sync_copy(dst_hbm.at[pl.ds(eb, LEK)], dstv)
        d1 = pltpu.async_copy(ea_hbm.at[pl.ds(eb, LEK)], ea_v, s1)
        d1.wait()

        def sbody(e, c2):
            row_v[e, pl.ds(16, 16)] = ea_v[e, pl.ds(0, 16)]
            return c2
        lax.fori_loop(0, LEK, sbody, 0)

        pltpu.sync_copy(row_v, acc.at[dstv], add=True)
        return c
    lax.fori_loop(0, LNCHUNK, chunk, 0)

    plsc.subcore_barrier()
    for t in range(CPS):
        cz = sid * CPS + t

        @pl.when(cz < NRC)
        def _():
            sl = pl.ds(cz * 80, 80)
            pltpu.sync_copy(acc.at[sl], out_hbm.at[cid, sl])


_sc_la = pl.kernel(
    _sc_la_body,
    out_type=jax.ShapeDtypeStruct((2, N, LA_W), jnp.float32),
    mesh=_MESH,
    compiler_params=_SC_PARAMS,
    scratch_types=[
        pltpu.VMEM((LEK,), jnp.int32),
        pltpu.VMEM((LEK, 16), jnp.float32),
        pltpu.VMEM((LEK, LA_W), jnp.float32),
        pltpu.VMEM((80, LA_W), jnp.float32),
        pltpu.VMEM_SHARED((N, LA_W), jnp.float32),
        pltpu.SemaphoreType.DMA,
    ],
)


def _mm(xs, ws, bias, act, bm):
    M = xs[0].shape[0]
    Ks = [x.shape[1] for x in xs]
    Nc = ws[0].shape[1]
    nx = len(xs)
    has_b = bias is not None

    def body(*refs):
        o = refs[-1]
        acc = None
        for i in range(nx):
            p = jnp.dot(refs[i][...], refs[nx + i][...],
                        preferred_element_type=jnp.float32)
            acc = p if acc is None else acc + p
        if has_b:
            acc = acc + refs[2 * nx][...]
        if act == "relu":
            acc = jnp.maximum(acc, 0.0)
        o[...] = acc

    in_specs = ([pl.BlockSpec((bm, k), lambda i: (i, 0)) for k in Ks]
                + [pl.BlockSpec((k, Nc), lambda i: (0, 0)) for k in Ks])
    if has_b:
        in_specs.append(pl.BlockSpec((1, Nc), lambda i: (0, 0)))
    args = list(xs) + list(ws) + ([bias] if has_b else [])
    return pl.pallas_call(
        body, grid=(M // bm,), in_specs=in_specs,
        out_specs=pl.BlockSpec((bm, Nc), lambda i: (i, 0)),
        out_shape=jax.ShapeDtypeStruct((M, Nc), jnp.float32))(*args)


def _fin(xl, xr, q, we_h, att, part_num, den4, bias, bm=200):
    with_q = q is not None

    def body(*refs):
        if with_q:
            xlr, xrr, qr, wer, attr, pr, ddr, br, o = refs
        else:
            xlr, xrr, attr, pr, ddr, br, o = refs
        z = xlr[...] + xrr[...]
        if with_q:
            qs = qr[0] + qr[1]
            cnt = jnp.maximum(qs[:, 0:1], 1.0)
            mea = qs[:, 16:32] / cnt
            z = z + jnp.dot(mea, wer[...], preferred_element_type=jnp.float32)
        m = jnp.maximum(z, 0.2 * z) * attr[...]
        ex = jnp.exp(jnp.sum(m, axis=1, keepdims=True))
        tot = ex * xlr[...] + pr[0] + pr[1]
        d = jnp.sum(ddr[:, 0, 0, :], axis=0)
        ii = lax.broadcasted_iota(jnp.int32, (bm, bm), 0)
        jj = lax.broadcasted_iota(jnp.int32, (bm, bm), 1)
        dmat = jnp.where(ii == jj, d[None, :], 0.0)
        dcol = jnp.sum(dmat, axis=1, keepdims=True) + ex + 1e-16
        out = tot / dcol + br[...]
        o[...] = jnp.where(out > 0, out, jnp.exp(out) - 1.0)

    nb = N // bm
    in_specs = [pl.BlockSpec((bm, 128), lambda i: (i, 0)),
                pl.BlockSpec((bm, 128), lambda i: (i, 0))]
    args = [xl, xr]
    if with_q:
        in_specs += [pl.BlockSpec((2, bm, LA_W), lambda i: (0, i, 0)),
                     pl.BlockSpec((16, 128), lambda i: (0, 0))]
        args += [q, we_h]
    in_specs += [pl.BlockSpec((1, 128), lambda i: (0, 0)),
                 pl.BlockSpec((2, bm, 128), lambda i: (0, i, 0)),
                 pl.BlockSpec((NW, 1, 1, bm), lambda i: (0, i, 0, 0)),
                 pl.BlockSpec((1, 128), lambda i: (0, 0))]
    args += [att, part_num, den4, bias]
    return pl.pallas_call(
        body, grid=(nb,), in_specs=in_specs,
        out_specs=pl.BlockSpec((bm, 128), lambda i: (i, 0)),
        out_shape=jax.ShapeDtypeStruct((N, 128), jnp.float32))(*args)


def _pool(h, batch3, bm=200):
    nb = N // bm

    def body(hr, br, mean_o, mx_o, sum_s, cnt_s, mx_s):
        i = pl.program_id(0)
        bblk = br[0, 0, :]
        iota = lax.broadcasted_iota(jnp.int32, (B, bm), 0)
        maskf = (bblk[None, :] == iota).astype(jnp.float32)
        psum = jnp.dot(maskf, hr[...], preferred_element_type=jnp.float32)
        pcnt = jnp.dot(maskf, jnp.ones((bm, 128), jnp.float32),
                       preferred_element_type=jnp.float32)
        pmx = jnp.full((B, 128), -1e30, jnp.float32)
        for j in range(bm // 8):
            sub = hr[pl.ds(j * 8, 8), :]
            msk = maskf[:, j * 8:(j + 1) * 8]
            cand = jnp.where(msk[:, :, None] > 0, sub[None, :, :], -1e30)
            pmx = jnp.maximum(pmx, jnp.max(cand, axis=1))

        @pl.when(i == 0)
        def _():
            sum_s[...] = psum
            cnt_s[...] = pcnt
            mx_s[...] = pmx

        @pl.when(i > 0)
        def _():
            sum_s[...] += psum
            cnt_s[...] += pcnt
            mx_s[...] = jnp.maximum(mx_s[...], pmx)

        @pl.when(i == nb - 1)
        def _():
            c = cnt_s[...]
            mean_o[...] = sum_s[...] / jnp.maximum(c, 1.0)
            mx_o[...] = jnp.where(c > 0, mx_s[...], 0.0)

    return pl.pallas_call(
        body, grid=(nb,),
        in_specs=[pl.BlockSpec((bm, 128), lambda i: (i, 0)),
                  pl.BlockSpec((1, 1, bm), lambda i: (i, 0, 0))],
        out_specs=[pl.BlockSpec((B, 128), lambda i: (0, 0)),
                   pl.BlockSpec((B, 128), lambda i: (0, 0))],
        out_shape=[jax.ShapeDtypeStruct((B, 128), jnp.float32),
                   jax.ShapeDtypeStruct((B, 128), jnp.float32)],
        scratch_shapes=[pltpu.VMEM((B, 128), jnp.float32),
                        pltpu.VMEM((B, 128), jnp.float32),
                        pltpu.VMEM((B, 128), jnp.float32)])(h, batch3)


def _heads(mean_a, mx_a, mean_b, mx_b, enz_a, enz_b, w):
    def body(ma, xa, mb, xb, ea, eb,
             ghw1, ghb1, ghw2, ghb2, ghw3, ghb3,
             phw1, phb1, phw2, phb2,
             agw1a, agw1b, agb1, agw2, agb2,
             fin_o, al_o):
        va = jnp.concatenate([ma[...], xa[...]], axis=1)
        vb = jnp.concatenate([mb[...], xb[...]], axis=1)
        comb = jnp.concatenate([va + vb, jnp.abs(va - vb), va * vb], axis=1)
        ec = jnp.concatenate([ea[...] + eb[...], jnp.abs(ea[...] - eb[...]),
                              ea[...] * eb[...]], axis=1)
        dot = lambda a, b: jnp.dot(a, b, preferred_element_type=jnp.float32)
        h1 = jnp.maximum(dot(comb, ghw1[...]) + ghb1[...], 0.0)
        h2 = jnp.maximum(dot(h1, ghw2[...]) + ghb2[...], 0.0)
        gl = dot(h2, ghw3[...]) + ghb3[...]
        p1 = jnp.maximum(dot(ec, phw1[...]) + phb1[...], 0.0)
        plg = dot(p1, phw2[...]) + phb2[...]
        g = jnp.maximum(dot(comb, agw1a[...]) + dot(ec, agw1b[...])
                        + agb1[...], 0.0)
        al = 1.0 / (1.0 + jnp.exp(-(dot(g, agw2[...]) + agb2[...])))
        fin_o[...] = al * gl + (1.0 - al) * plg
        al_o[...] = al

    return pl.pallas_call(
        body,
        out_shape=[jax.ShapeDtypeStruct((B, 1), jnp.float32),
                   jax.ShapeDtypeStruct((B, 1), jnp.float32)],
    )(mean_a, mx_a, mean_b, mx_b, enz_a, enz_b, *w)


def kernel(x_a, edge_index_a, edge_attr_a, batch_a, enzyme_a,
           x_b, edge_index_b, edge_attr_b, batch_b, enzyme_b, params):
    p = params

    s_enc = p['enc_bn_g'] / jnp.sqrt(p['enc_bn_v'] + 1e-5)
    enc_w = p['enc_w'] * s_enc[None, :]
    enc_b = ((p['enc_b'] - p['enc_bn_m']) * s_enc + p['enc_bn_b']).reshape(1, -1)

    s_gh = p['gh_bn_g'] / jnp.sqrt(p['gh_bn_v'] + 1e-5)
    ghw1 = p['gh_w1'] * s_gh[None, :]
    ghb1 = ((p['gh_b1'] - p['gh_bn_m']) * s_gh + p['gh_bn_b']).reshape(1, -1)

    c1 = []
    for hh in range(HEADS):
        sl = slice(hh * 128, (hh + 1) * 128)
        c1.append(dict(
            wl=p['c1_wl'][:, sl], bl=p['c1_bl'][sl].reshape(1, -1),
            wr=p['c1_wr'][:, sl], br=p['c1_br'][sl].reshape(1, -1),
            we=p['c1_we'][:, sl], att=p['c1_att'][hh].reshape(1, -1),
            bias=p['c1_bias'][sl].reshape(1, -1)))
    c2_wl = [p['c2_wl'][i * 128:(i + 1) * 128, :] for i in range(HEADS)]
    c2_wr = [p['c2_wr'][i * 128:(i + 1) * 128, :] for i in range(HEADS)]

    def conv(xl, xr, eeh, q, we_h, att, bias, src, dst):
        part_num, part_den = _sc_edge(xl, xr, eeh, src, dst,
                                      jnp.pad(att.reshape(-1), (0, 16)))
        den4 = part_den.reshape(NW, N // 200, 1, 200)
        return _fin(xl, xr, q, we_h, att, part_num, den4, bias)

    def arm(x, ei, ea, batch):
        src, dst = ei[0], ei[1]
        h0 = _mm([x], [enc_w], enc_b, "relu", 200)
        q = _sc_la(ea, dst)
        outs1 = []
        for hh in range(HEADS):
            c = c1[hh]
            xl = _mm([h0], [c['wl']], c['bl'], None, 200)
            xr = _mm([h0], [c['wr']], c['br'], None, 200)
            eeh = _mm([ea], [c['we']], None, None, 256)
            outs1.append(conv(xl, xr, eeh, q, c['we'], c['att'], c['bias'],
                              src, dst))
        xl2 = _mm(outs1, c2_wl, p['c2_bl'].reshape(1, -1), None, 200)
        xr2 = _mm(outs1, c2_wr, p['c2_br'].reshape(1, -1), None, 200)
        ee2 = _mm([ea], [p['c2_we']], None, None, 256)
        att2 = p['c2_att'].reshape(1, -1)
        h2 = conv(xl2, xr2, ee2, None, None, att2,
                  p['c2_bias'].reshape(1, -1), src, dst)
        batch3 = batch.reshape(N // 200, 1, 200)
        return _pool(h2, batch3)

    mean_a, mx_a = arm(x_a, edge_index_a, edge_attr_a, batch_a)
    mean_b, mx_b = arm(x_b, edge_index_b, edge_attr_b, batch_b)

    w = [ghw1, ghb1, p['gh_w2'], p['gh_b2'].reshape(1, -1),
         p['gh_w3'], p['gh_b3'].reshape(1, -1),
         p['ph_w1'], p['ph_b1'].reshape(1, -1),
         p['ph_w2'], p['ph_b2'].reshape(1, -1),
         p['ag_w1'][:768], p['ag_w1'][768:], p['ag_b1'].reshape(1, -1),
         p['ag_w2'], p['ag_b2'].reshape(1, -1)]
    final, alpha = _heads(mean_a, mx_a, mean_b, mx_b, enzyme_a, enzyme_b, w)
    return final, alpha
